# baseline (speedup 1.0000x reference)
import jax
import jax.numpy as jnp
from jax.experimental import pallas as pl
from jax.experimental.pallas import tpu as pltpu

N_BONDS = 160000
N_ATOMS = 10000
HIDDEN = 1024
DEPTH = 8

BB = 1000
AB = 2000


def _k0_body(fb_ref, wi_ref, bi_ref, inp_ref, m_ref):
    x = jnp.dot(fb_ref[...], wi_ref[...], preferred_element_type=jnp.float32)
    x = x + bi_ref[...]
    inp_ref[...] = x
    m_ref[...] = jnp.maximum(x, 0.0)


def _k0(f_bonds, W_i, b_i):
    nfb = f_bonds.shape[1]
    grid = (N_BONDS // BB,)
    return pl.pallas_call(
        _k0_body,
        grid=grid,
        in_specs=[
            pl.BlockSpec((BB, nfb), lambda i: (i, 0)),
            pl.BlockSpec((nfb, HIDDEN), lambda i: (0, 0)),
            pl.BlockSpec((1, HIDDEN), lambda i: (0, 0)),
        ],
        out_specs=[
            pl.BlockSpec((BB, HIDDEN), lambda i: (i, 0)),
            pl.BlockSpec((BB, HIDDEN), lambda i: (i, 0)),
        ],
        out_shape=[
            jax.ShapeDtypeStruct((N_BONDS, HIDDEN), jnp.float32),
            jax.ShapeDtypeStruct((N_BONDS, HIDDEN), jnp.float32),
        ],
    )(f_bonds, W_i, b_i.reshape(1, HIDDEN))


def _k3_body(d_ref, inp_ref, wh_ref, bh_ref, m_ref):
    x = jnp.dot(d_ref[...], wh_ref[...], preferred_element_type=jnp.float32)
    m_ref[...] = jnp.maximum(inp_ref[...] + x + bh_ref[...], 0.0)


def _k3(D, inp, W_h, b_h):
    grid = (N_BONDS // BB,)
    return pl.pallas_call(
        _k3_body,
        grid=grid,
        in_specs=[
            pl.BlockSpec((BB, HIDDEN), lambda i: (i, 0)),
            pl.BlockSpec((BB, HIDDEN), lambda i: (i, 0)),
            pl.BlockSpec((HIDDEN, HIDDEN), lambda i: (0, 0)),
            pl.BlockSpec((1, HIDDEN), lambda i: (0, 0)),
        ],
        out_specs=pl.BlockSpec((BB, HIDDEN), lambda i: (i, 0)),
        out_shape=jax.ShapeDtypeStruct((N_BONDS, HIDDEN), jnp.float32),
    )(D, inp, W_h, b_h.reshape(1, HIDDEN))


def _k4_body(fa_ref, s_ref, woa_ref, wom_ref, bo_ref, pool_ref, out_ref):
    h = jnp.dot(fa_ref[...], woa_ref[...], preferred_element_type=jnp.float32)
    h += jnp.dot(s_ref[...], wom_ref[...], preferred_element_type=jnp.float32)
    h = jnp.maximum(h + bo_ref[...], 0.0)
    contrib = jax.lax.dot_general(
        pool_ref[...], h, (((0,), (0,)), ((), ())),
        preferred_element_type=jnp.float32)

    @pl.when(pl.program_id(0) == 0)
    def _():
        out_ref[...] = jnp.zeros_like(out_ref)

    out_ref[...] += contrib


def _k4(f_atoms, S, W_o, b_o, pool):
    nfa = f_atoms.shape[1]
    n_mols = pool.shape[1]
    W_oa = W_o[:nfa]
    W_om = W_o[nfa:]
    grid = (N_ATOMS // AB,)
    return pl.pallas_call(
        _k4_body,
        grid=grid,
        in_specs=[
            pl.BlockSpec((AB, nfa), lambda i: (i, 0)),
            pl.BlockSpec((AB, HIDDEN), lambda i: (i, 0)),
            pl.BlockSpec((nfa, HIDDEN), lambda i: (0, 0)),
            pl.BlockSpec((HIDDEN, HIDDEN), lambda i: (0, 0)),
            pl.BlockSpec((1, HIDDEN), lambda i: (0, 0)),
            pl.BlockSpec((AB, n_mols), lambda i: (i, 0)),
        ],
        out_specs=pl.BlockSpec((n_mols, HIDDEN), lambda i: (0, 0)),
        out_shape=jax.ShapeDtypeStruct((n_mols, HIDDEN), jnp.float32),
    )(f_atoms, S, W_oa, W_om, b_o.reshape(1, HIDDEN), pool)


def _segsum(M, a2b):
    return M[a2b].sum(axis=1)


def kernel(f_atoms, f_bonds, a2b, b2a, b2revb, a_scope, W_i, b_i, W_h, b_h, W_o, b_o):
    n_mols = a_scope.shape[0]
    sizes = a_scope[:, 1]
    seg_ids = jnp.repeat(jnp.arange(n_mols, dtype=jnp.int32), sizes,
                         total_repeat_length=N_ATOMS)
    pool = (seg_ids[:, None] == jnp.arange(n_mols, dtype=jnp.int32)[None, :])
    pool = pool.astype(jnp.float32) / sizes[None, :].astype(jnp.float32)

    inp, M = _k0(f_bonds, W_i, b_i)
    for _ in range(DEPTH - 1):
        S = _segsum(M, a2b)
        D = S[b2a] - M[b2revb]
        M = _k3(D, inp, W_h, b_h)
    S = _segsum(M, a2b)
    return _k4(f_atoms, S, W_o, b_o, pool)


# speedup vs baseline: 1.1646x; 1.1646x over previous
import functools

import jax
import jax.numpy as jnp
from jax import lax
from jax.experimental import pallas as pl
from jax.experimental.pallas import tpu as pltpu
from jax.experimental.pallas import tpu_sc as plsc

N_BONDS = 160000
N_ATOMS = 10000
HIDDEN = 1024
DEPTH = 8

NW = 32
N_ATOMS_PAD = 10240
APW = N_ATOMS_PAD // NW
BPW = N_BONDS // NW
SEG_C = 2
DIF_C = 8


def _sum16(rows_ref, stage_ref, n_atoms):
    @pl.loop(0, HIDDEN, step=16)
    def _(c):
        for a in range(n_atoms):
            acc = rows_ref[a * 16, pl.ds(c, 16)]
            for i in range(1, 16):
                acc = acc + rows_ref[a * 16 + i, pl.ds(c, 16)]
            stage_ref[a, pl.ds(c, 16)] = acc


def _sc_segsum(M, a2b_flat):
    mesh = plsc.VectorSubcoreMesh(core_axis_name="c", subcore_axis_name="s")
    n_pairs = APW // SEG_C // 2

    @functools.partial(
        pl.kernel, mesh=mesh,
        out_type=jax.ShapeDtypeStruct((N_ATOMS_PAD, HIDDEN), jnp.float32),
        scratch_types=[
            pltpu.VMEM((APW * 16,), jnp.int32),
            pltpu.VMEM((SEG_C * 16, HIDDEN), jnp.float32),
            pltpu.VMEM((SEG_C * 16, HIDDEN), jnp.float32),
            pltpu.VMEM((SEG_C, HIDDEN), jnp.float32),
            pltpu.VMEM((SEG_C, HIDDEN), jnp.float32),
            pltpu.SemaphoreType.DMA,
            pltpu.SemaphoreType.DMA,
            pltpu.SemaphoreType.DMA,
            pltpu.SemaphoreType.DMA,
        ],
    )
    def k(m_hbm, idx_hbm, out_hbm, idx_v, r0, r1, st0, st1, g0, g1, s0, s1):
        wid = lax.axis_index("s") * 2 + lax.axis_index("c")
        pltpu.sync_copy(idx_hbm.at[pl.ds(wid * APW * 16, APW * 16)], idx_v)
        abase = wid * APW

        def gather(j, dst, sem):
            return pltpu.async_copy(
                m_hbm.at[idx_v.at[pl.ds(j * (SEG_C * 16), SEG_C * 16)]], dst, sem)

        def gwait(j, dst, sem):
            pltpu.make_async_copy(
                m_hbm.at[idx_v.at[pl.ds(j * (SEG_C * 16), SEG_C * 16)]], dst,
                sem).wait()

        def store(j, src, sem):
            return pltpu.async_copy(
                src, out_hbm.at[pl.ds(abase + j * SEG_C, SEG_C)], sem)

        def swait(src, sem):
            pltpu.make_async_copy(
                src, out_hbm.at[pl.ds(abase, SEG_C)], sem).wait()

        gather(0, r0, g0)
        gather(1, r1, g1)

        @pl.loop(0, n_pairs)
        def _(kk):
            j0 = 2 * kk
            j1 = j0 + 1

            @pl.when(kk > 0)
            def _():
                swait(st1, s1)
                gather(j1, r1, g1)

            gwait(j0, r0, g0)
            _sum16(r0, st0, SEG_C)
            store(j0, st0, s0)

            gwait(j1, r1, g1)
            _sum16(r1, st1, SEG_C)
            store(j1, st1, s1)

            @pl.when(kk < n_pairs - 1)
            def _():
                swait(st0, s0)
                gather(j0 + 2, r0, g0)

        swait(st0, s0)
        swait(st1, s1)

    return k(M, a2b_flat)

BB = 1000
AB = 2000


def _sc_diff(S, M, b2a, b2revb):
    mesh = plsc.VectorSubcoreMesh(core_axis_name="c", subcore_axis_name="s")
    n_chunks = BPW // DIF_C
    n_pairs = n_chunks // 2

    @functools.partial(
        pl.kernel, mesh=mesh,
        out_type=jax.ShapeDtypeStruct((N_BONDS, HIDDEN), jnp.float32),
        scratch_types=[
            pltpu.VMEM((BPW,), jnp.int32),
            pltpu.VMEM((BPW,), jnp.int32),
            pltpu.VMEM((DIF_C, HIDDEN), jnp.float32),
            pltpu.VMEM((DIF_C, HIDDEN), jnp.float32),
            pltpu.VMEM((DIF_C, HIDDEN), jnp.float32),
            pltpu.VMEM((DIF_C, HIDDEN), jnp.float32),
            pltpu.SemaphoreType.DMA,
            pltpu.SemaphoreType.DMA,
            pltpu.SemaphoreType.DMA,
            pltpu.SemaphoreType.DMA,
            pltpu.SemaphoreType.DMA,
            pltpu.SemaphoreType.DMA,
        ],
    )
    def k(s_hbm, m_hbm, ia_hbm, ir_hbm, out_hbm,
          ia_v, ir_v, sa0, sa1, mb0, mb1, ga0, ga1, gm0, gm1, ss0, ss1):
        wid = lax.axis_index("s") * 2 + lax.axis_index("c")
        pltpu.sync_copy(ia_hbm.at[pl.ds(wid * BPW, BPW)], ia_v)
        pltpu.sync_copy(ir_hbm.at[pl.ds(wid * BPW, BPW)], ir_v)
        bbase = wid * BPW

        def gathers(j, sdst, mdst, gsa, gsm):
            pltpu.async_copy(s_hbm.at[ia_v.at[pl.ds(j * DIF_C, DIF_C)]], sdst, gsa)
            pltpu.async_copy(m_hbm.at[ir_v.at[pl.ds(j * DIF_C, DIF_C)]], mdst, gsm)

        def gwaits(j, sdst, mdst, gsa, gsm):
            pltpu.make_async_copy(
                s_hbm.at[ia_v.at[pl.ds(j * DIF_C, DIF_C)]], sdst, gsa).wait()
            pltpu.make_async_copy(
                m_hbm.at[ir_v.at[pl.ds(j * DIF_C, DIF_C)]], mdst, gsm).wait()

        def sub_inplace(sdst, mdst):
            @pl.loop(0, HIDDEN, step=16)
            def _(c):
                for r in range(DIF_C):
                    sdst[r, pl.ds(c, 16)] = (
                        sdst[r, pl.ds(c, 16)] - mdst[r, pl.ds(c, 16)])

        def store(j, src, sem):
            pltpu.async_copy(src, out_hbm.at[pl.ds(bbase + j * DIF_C, DIF_C)], sem)

        def swait(src, sem):
            pltpu.make_async_copy(
                src, out_hbm.at[pl.ds(bbase, DIF_C)], sem).wait()

        gathers(0, sa0, mb0, ga0, gm0)
        gathers(1, sa1, mb1, ga1, gm1)

        @pl.loop(0, n_pairs)
        def _(kk):
            j0 = 2 * kk
            j1 = j0 + 1

            @pl.when(kk > 0)
            def _():
                swait(sa1, ss1)
                gathers(j1, sa1, mb1, ga1, gm1)

            gwaits(j0, sa0, mb0, ga0, gm0)
            sub_inplace(sa0, mb0)
            store(j0, sa0, ss0)

            gwaits(j1, sa1, mb1, ga1, gm1)
            sub_inplace(sa1, mb1)
            store(j1, sa1, ss1)

            swait(sa0, ss0)

            @pl.when(kk < n_pairs - 1)
            def _():
                gathers(j0 + 2, sa0, mb0, ga0, gm0)

        jlast = n_chunks - 1
        gathers(jlast, sa0, mb0, ga0, gm0)
        gwaits(jlast, sa0, mb0, ga0, gm0)
        sub_inplace(sa0, mb0)
        store(jlast, sa0, ss0)
        swait(sa0, ss0)
        swait(sa1, ss1)

    return k(S, M, b2a, b2revb)


def _k0_body(fb_ref, wi_ref, bi_ref, inp_ref, m_ref):
    x = jnp.dot(fb_ref[...], wi_ref[...], preferred_element_type=jnp.float32)
    x = x + bi_ref[...]
    inp_ref[...] = x
    m_ref[...] = jnp.maximum(x, 0.0)


def _k0(f_bonds, W_i, b_i):
    nfb = f_bonds.shape[1]
    grid = (N_BONDS // BB,)
    return pl.pallas_call(
        _k0_body,
        grid=grid,
        in_specs=[
            pl.BlockSpec((BB, nfb), lambda i: (i, 0)),
            pl.BlockSpec((nfb, HIDDEN), lambda i: (0, 0)),
            pl.BlockSpec((1, HIDDEN), lambda i: (0, 0)),
        ],
        out_specs=[
            pl.BlockSpec((BB, HIDDEN), lambda i: (i, 0)),
            pl.BlockSpec((BB, HIDDEN), lambda i: (i, 0)),
        ],
        out_shape=[
            jax.ShapeDtypeStruct((N_BONDS, HIDDEN), jnp.float32),
            jax.ShapeDtypeStruct((N_BONDS, HIDDEN), jnp.float32),
        ],
    )(f_bonds, W_i, b_i.reshape(1, HIDDEN))


def _k3_body(d_ref, inp_ref, wh_ref, bh_ref, m_ref):
    x = jnp.dot(d_ref[...], wh_ref[...], preferred_element_type=jnp.float32)
    m_ref[...] = jnp.maximum(inp_ref[...] + x + bh_ref[...], 0.0)


def _k3(D, inp, W_h, b_h):
    grid = (N_BONDS // BB,)
    return pl.pallas_call(
        _k3_body,
        grid=grid,
        in_specs=[
            pl.BlockSpec((BB, HIDDEN), lambda i: (i, 0)),
            pl.BlockSpec((BB, HIDDEN), lambda i: (i, 0)),
            pl.BlockSpec((HIDDEN, HIDDEN), lambda i: (0, 0)),
            pl.BlockSpec((1, HIDDEN), lambda i: (0, 0)),
        ],
        out_specs=pl.BlockSpec((BB, HIDDEN), lambda i: (i, 0)),
        out_shape=jax.ShapeDtypeStruct((N_BONDS, HIDDEN), jnp.float32),
    )(D, inp, W_h, b_h.reshape(1, HIDDEN))


def _k4_body(fa_ref, s_ref, woa_ref, wom_ref, bo_ref, pool_ref, out_ref):
    h = jnp.dot(fa_ref[...], woa_ref[...], preferred_element_type=jnp.float32)
    h += jnp.dot(s_ref[...], wom_ref[...], preferred_element_type=jnp.float32)
    h = jnp.maximum(h + bo_ref[...], 0.0)
    contrib = jax.lax.dot_general(
        pool_ref[...], h, (((0,), (0,)), ((), ())),
        preferred_element_type=jnp.float32)

    @pl.when(pl.program_id(0) == 0)
    def _():
        out_ref[...] = jnp.zeros_like(out_ref)

    out_ref[...] += contrib


def _k4(f_atoms, S, W_o, b_o, pool):
    nfa = f_atoms.shape[1]
    n_mols = pool.shape[1]
    W_oa = W_o[:nfa]
    W_om = W_o[nfa:]
    grid = (N_ATOMS // AB,)
    return pl.pallas_call(
        _k4_body,
        grid=grid,
        in_specs=[
            pl.BlockSpec((AB, nfa), lambda i: (i, 0)),
            pl.BlockSpec((AB, HIDDEN), lambda i: (i, 0)),
            pl.BlockSpec((nfa, HIDDEN), lambda i: (0, 0)),
            pl.BlockSpec((HIDDEN, HIDDEN), lambda i: (0, 0)),
            pl.BlockSpec((1, HIDDEN), lambda i: (0, 0)),
            pl.BlockSpec((AB, n_mols), lambda i: (i, 0)),
        ],
        out_specs=pl.BlockSpec((n_mols, HIDDEN), lambda i: (0, 0)),
        out_shape=jax.ShapeDtypeStruct((n_mols, HIDDEN), jnp.float32),
    )(f_atoms, S, W_oa, W_om, b_o.reshape(1, HIDDEN), pool)


def kernel(f_atoms, f_bonds, a2b, b2a, b2revb, a_scope, W_i, b_i, W_h, b_h, W_o, b_o):
    n_mols = a_scope.shape[0]
    sizes = a_scope[:, 1]
    seg_ids = jnp.repeat(jnp.arange(n_mols, dtype=jnp.int32), sizes,
                         total_repeat_length=N_ATOMS)
    pool = (seg_ids[:, None] == jnp.arange(n_mols, dtype=jnp.int32)[None, :])
    pool = pool.astype(jnp.float32) / sizes[None, :].astype(jnp.float32)

    n_pad = N_ATOMS_PAD - N_ATOMS
    pad_idx = (jnp.arange(n_pad * 16, dtype=jnp.int32) * 37) % N_BONDS
    a2b_flat = jnp.concatenate([a2b.reshape(-1), pad_idx])

    inp, M = _k0(f_bonds, W_i, b_i)
    for _ in range(DEPTH - 1):
        S = _sc_segsum(M, a2b_flat)
        D = _sc_diff(S, M, b2a, b2revb)
        M = _k3(D, inp, W_h, b_h)
    S = _sc_segsum(M, a2b_flat)
    return _k4(f_atoms, S, W_o, b_o, pool)


# speedup vs baseline: 1.7182x; 1.4754x over previous
import dataclasses
import functools

import jax
import jax.numpy as jnp
from jax import lax
from jax.experimental import pallas as pl
from jax.experimental.pallas import tpu as pltpu
from jax.experimental.pallas import tpu_sc as plsc

N_BONDS = 160000
N_ATOMS = 10000
HIDDEN = 1024
HALF = HIDDEN // 2
DEPTH = 8

NW = 32
N_ATOMS_PAD = 10240
APW = N_ATOMS_PAD // NW
BPW = N_BONDS // NW
SEG_C = 2
DIF_C = 8

BB = 2000
AB = 2000


def _sc_compiler_params():
    cp = pltpu.CompilerParams()
    if "needs_layout_passes" in pltpu.CompilerParams.__dataclass_fields__:
        cp = dataclasses.replace(cp, needs_layout_passes=False)
    return cp


_MASK_HI = -65536


def _tc_pack(x):
    lo = lax.bitcast_convert_type(x[:, :HALF], jnp.int32)
    hi = lax.bitcast_convert_type(x[:, HALF:], jnp.int32)
    lo_w = lax.shift_right_logical(lo + 32768, 16)
    hi_w = (hi + 32768) & _MASK_HI
    return lo_w | hi_w


def _tc_unpack_bf16(w):
    lo = lax.bitcast_convert_type(lax.shift_left(w, 16), jnp.float32)
    hi = lax.bitcast_convert_type(w & _MASK_HI, jnp.float32)
    return lo.astype(jnp.bfloat16), hi.astype(jnp.bfloat16)


def _sum16(rows_ref, stage_ref, n_atoms):
    @pl.loop(0, HALF, step=16)
    def _(c):
        for a in range(n_atoms):
            acc_a, acc_b = plsc.unpack(
                plsc.bitcast(rows_ref[a * 16, pl.ds(c, 16)], jnp.bfloat16),
                format=plsc.PackFormat.INTERLEAVED,
                preferred_element_type=jnp.float32)
            for i in range(1, 16):
                x_a, x_b = plsc.unpack(
                    plsc.bitcast(rows_ref[a * 16 + i, pl.ds(c, 16)],
                                 jnp.bfloat16),
                    format=plsc.PackFormat.INTERLEAVED,
                    preferred_element_type=jnp.float32)
                acc_a = acc_a + x_a
                acc_b = acc_b + x_b
            stage_ref[a, pl.ds(c, 16)] = plsc.bitcast(
                plsc.pack(acc_a, acc_b, format=plsc.PackFormat.INTERLEAVED),
                jnp.int32)


def _sc_segsum(M, a2b_flat):
    mesh = plsc.VectorSubcoreMesh(core_axis_name="c", subcore_axis_name="s")
    n_pairs = APW // SEG_C // 2

    @functools.partial(
        pl.kernel, mesh=mesh, compiler_params=_sc_compiler_params(),
        out_type=jax.ShapeDtypeStruct((N_ATOMS_PAD, HALF), jnp.int32),
        scratch_types=[
            pltpu.VMEM((APW * 16,), jnp.int32),
            pltpu.VMEM((SEG_C * 16, HALF), jnp.int32),
            pltpu.VMEM((SEG_C * 16, HALF), jnp.int32),
            pltpu.VMEM((SEG_C, HALF), jnp.int32),
            pltpu.VMEM((SEG_C, HALF), jnp.int32),
            pltpu.SemaphoreType.DMA,
            pltpu.SemaphoreType.DMA,
            pltpu.SemaphoreType.DMA,
            pltpu.SemaphoreType.DMA,
        ],
    )
    def k(m_hbm, idx_hbm, out_hbm, idx_v, r0, r1, st0, st1, g0, g1, s0, s1):
        wid = lax.axis_index("s") * 2 + lax.axis_index("c")
        pltpu.sync_copy(idx_hbm.at[pl.ds(wid * APW * 16, APW * 16)], idx_v)
        abase = wid * APW

        def gather(j, dst, sem):
            return pltpu.async_copy(
                m_hbm.at[idx_v.at[pl.ds(j * (SEG_C * 16), SEG_C * 16)]], dst, sem)

        def gwait(j, dst, sem):
            pltpu.make_async_copy(
                m_hbm.at[idx_v.at[pl.ds(j * (SEG_C * 16), SEG_C * 16)]], dst,
                sem).wait()

        def store(j, src, sem):
            return pltpu.async_copy(
                src, out_hbm.at[pl.ds(abase + j * SEG_C, SEG_C)], sem)

        def swait(src, sem):
            pltpu.make_async_copy(
                src, out_hbm.at[pl.ds(abase, SEG_C)], sem).wait()

        gather(0, r0, g0)
        gather(1, r1, g1)

        @pl.loop(0, n_pairs)
        def _(kk):
            j0 = 2 * kk
            j1 = j0 + 1

            @pl.when(kk > 0)
            def _():
                swait(st1, s1)
                gather(j1, r1, g1)

            gwait(j0, r0, g0)
            _sum16(r0, st0, SEG_C)
            store(j0, st0, s0)

            gwait(j1, r1, g1)
            _sum16(r1, st1, SEG_C)
            store(j1, st1, s1)

            @pl.when(kk < n_pairs - 1)
            def _():
                swait(st0, s0)
                gather(j0 + 2, r0, g0)

        swait(st0, s0)
        swait(st1, s1)

    return k(M, a2b_flat)


def _sc_diff(S, M, b2a, b2revb):
    mesh = plsc.VectorSubcoreMesh(core_axis_name="c", subcore_axis_name="s")
    n_chunks = BPW // DIF_C
    n_pairs = n_chunks // 2

    @functools.partial(
        pl.kernel, mesh=mesh, compiler_params=_sc_compiler_params(),
        out_type=jax.ShapeDtypeStruct((N_BONDS, HALF), jnp.int32),
        scratch_types=[
            pltpu.VMEM((BPW,), jnp.int32),
            pltpu.VMEM((BPW,), jnp.int32),
            pltpu.VMEM((DIF_C, HALF), jnp.int32),
            pltpu.VMEM((DIF_C, HALF), jnp.int32),
            pltpu.VMEM((DIF_C, HALF), jnp.int32),
            pltpu.VMEM((DIF_C, HALF), jnp.int32),
            pltpu.SemaphoreType.DMA,
            pltpu.SemaphoreType.DMA,
            pltpu.SemaphoreType.DMA,
            pltpu.SemaphoreType.DMA,
            pltpu.SemaphoreType.DMA,
            pltpu.SemaphoreType.DMA,
        ],
    )
    def k(s_hbm, m_hbm, ia_hbm, ir_hbm, out_hbm,
          ia_v, ir_v, sa0, sa1, mb0, mb1, ga0, ga1, gm0, gm1, ss0, ss1):
        wid = lax.axis_index("s") * 2 + lax.axis_index("c")
        pltpu.sync_copy(ia_hbm.at[pl.ds(wid * BPW, BPW)], ia_v)
        pltpu.sync_copy(ir_hbm.at[pl.ds(wid * BPW, BPW)], ir_v)
        bbase = wid * BPW

        def gathers(j, sdst, mdst, gsa, gsm):
            pltpu.async_copy(s_hbm.at[ia_v.at[pl.ds(j * DIF_C, DIF_C)]], sdst, gsa)
            pltpu.async_copy(m_hbm.at[ir_v.at[pl.ds(j * DIF_C, DIF_C)]], mdst, gsm)

        def gwaits(j, sdst, mdst, gsa, gsm):
            pltpu.make_async_copy(
                s_hbm.at[ia_v.at[pl.ds(j * DIF_C, DIF_C)]], sdst, gsa).wait()
            pltpu.make_async_copy(
                m_hbm.at[ir_v.at[pl.ds(j * DIF_C, DIF_C)]], mdst, gsm).wait()

        def sub_inplace(sdst, mdst):
            @pl.loop(0, HALF, step=16)
            def _(c):
                for r in range(DIF_C):
                    a = plsc.bitcast(sdst[r, pl.ds(c, 16)], jnp.bfloat16)
                    b = plsc.bitcast(mdst[r, pl.ds(c, 16)], jnp.bfloat16)
                    sdst[r, pl.ds(c, 16)] = plsc.bitcast(a - b, jnp.int32)

        def store(j, src, sem):
            pltpu.async_copy(src, out_hbm.at[pl.ds(bbase + j * DIF_C, DIF_C)], sem)

        def swait(src, sem):
            pltpu.make_async_copy(
                src, out_hbm.at[pl.ds(bbase, DIF_C)], sem).wait()

        gathers(0, sa0, mb0, ga0, gm0)
        gathers(1, sa1, mb1, ga1, gm1)

        @pl.loop(0, n_pairs)
        def _(kk):
            j0 = 2 * kk
            j1 = j0 + 1

            @pl.when(kk > 0)
            def _():
                swait(sa1, ss1)
                gathers(j1, sa1, mb1, ga1, gm1)

            gwaits(j0, sa0, mb0, ga0, gm0)
            sub_inplace(sa0, mb0)
            store(j0, sa0, ss0)

            gwaits(j1, sa1, mb1, ga1, gm1)
            sub_inplace(sa1, mb1)
            store(j1, sa1, ss1)

            swait(sa0, ss0)

            @pl.when(kk < n_pairs - 1)
            def _():
                gathers(j0 + 2, sa0, mb0, ga0, gm0)

        jlast = n_chunks - 1
        gathers(jlast, sa0, mb0, ga0, gm0)
        gwaits(jlast, sa0, mb0, ga0, gm0)
        sub_inplace(sa0, mb0)
        store(jlast, sa0, ss0)
        swait(sa0, ss0)
        swait(sa1, ss1)

    return k(S, M, b2a, b2revb)


def _k0_body(fb_ref, wi_ref, bi_ref, inp_ref, m_ref):
    x = jnp.dot(fb_ref[...], wi_ref[...], preferred_element_type=jnp.float32)
    x = x + bi_ref[...]
    inp_ref[...] = x
    m_ref[...] = _tc_pack(jnp.maximum(x, 0.0))


def _k0(f_bonds, W_i, b_i):
    nfb = f_bonds.shape[1]
    grid = (N_BONDS // BB,)
    return pl.pallas_call(
        _k0_body,
        grid=grid,
        in_specs=[
            pl.BlockSpec((BB, nfb), lambda i: (i, 0)),
            pl.BlockSpec((nfb, HIDDEN), lambda i: (0, 0)),
            pl.BlockSpec((1, HIDDEN), lambda i: (0, 0)),
        ],
        out_specs=[
            pl.BlockSpec((BB, HIDDEN), lambda i: (i, 0)),
            pl.BlockSpec((BB, HALF), lambda i: (i, 0)),
        ],
        out_shape=[
            jax.ShapeDtypeStruct((N_BONDS, HIDDEN), jnp.float32),
            jax.ShapeDtypeStruct((N_BONDS, HALF), jnp.int32),
        ],
    )(f_bonds, W_i, b_i.reshape(1, HIDDEN))


def _k3_body(d_ref, inp_ref, wlo_ref, whi_ref, bh_ref, m_ref):
    lo, hi = _tc_unpack_bf16(d_ref[...])
    x = jnp.dot(lo, wlo_ref[...], preferred_element_type=jnp.float32)
    x += jnp.dot(hi, whi_ref[...], preferred_element_type=jnp.float32)
    m_ref[...] = _tc_pack(jnp.maximum(inp_ref[...] + x + bh_ref[...], 0.0))


def _k3(D, inp, W_lo, W_hi, b_h):
    grid = (N_BONDS // BB,)
    return pl.pallas_call(
        _k3_body,
        grid=grid,
        in_specs=[
            pl.BlockSpec((BB, HALF), lambda i: (i, 0)),
            pl.BlockSpec((BB, HIDDEN), lambda i: (i, 0)),
            pl.BlockSpec((HALF, HIDDEN), lambda i: (0, 0)),
            pl.BlockSpec((HALF, HIDDEN), lambda i: (0, 0)),
            pl.BlockSpec((1, HIDDEN), lambda i: (0, 0)),
        ],
        out_specs=pl.BlockSpec((BB, HALF), lambda i: (i, 0)),
        out_shape=jax.ShapeDtypeStruct((N_BONDS, HALF), jnp.int32),
    )(D, inp, W_lo, W_hi, b_h.reshape(1, HIDDEN))


def _k4_body(fa_ref, s_ref, woa_ref, wlo_ref, whi_ref, bo_ref, pool_ref,
             out_ref):
    h = jnp.dot(fa_ref[...], woa_ref[...], preferred_element_type=jnp.float32)
    lo, hi = _tc_unpack_bf16(s_ref[...])
    h += jnp.dot(lo, wlo_ref[...], preferred_element_type=jnp.float32)
    h += jnp.dot(hi, whi_ref[...], preferred_element_type=jnp.float32)
    h = jnp.maximum(h + bo_ref[...], 0.0)
    contrib = jax.lax.dot_general(
        pool_ref[...], h, (((0,), (0,)), ((), ())),
        preferred_element_type=jnp.float32)

    @pl.when(pl.program_id(0) == 0)
    def _():
        out_ref[...] = jnp.zeros_like(out_ref)

    out_ref[...] += contrib


def _k4(f_atoms, S, W_o, b_o, pool):
    nfa = f_atoms.shape[1]
    n_mols = pool.shape[1]
    W_oa = W_o[:nfa]
    W_lo = W_o[nfa:nfa + HALF].astype(jnp.bfloat16)
    W_hi = W_o[nfa + HALF:].astype(jnp.bfloat16)
    grid = (N_ATOMS // AB,)
    return pl.pallas_call(
        _k4_body,
        grid=grid,
        in_specs=[
            pl.BlockSpec((AB, nfa), lambda i: (i, 0)),
            pl.BlockSpec((AB, HALF), lambda i: (i, 0)),
            pl.BlockSpec((nfa, HIDDEN), lambda i: (0, 0)),
            pl.BlockSpec((HALF, HIDDEN), lambda i: (0, 0)),
            pl.BlockSpec((HALF, HIDDEN), lambda i: (0, 0)),
            pl.BlockSpec((1, HIDDEN), lambda i: (0, 0)),
            pl.BlockSpec((AB, n_mols), lambda i: (i, 0)),
        ],
        out_specs=pl.BlockSpec((n_mols, HIDDEN), lambda i: (0, 0)),
        out_shape=jax.ShapeDtypeStruct((n_mols, HIDDEN), jnp.float32),
    )(f_atoms, S, W_oa, W_lo, W_hi, b_o.reshape(1, HIDDEN), pool)


def kernel(f_atoms, f_bonds, a2b, b2a, b2revb, a_scope, W_i, b_i, W_h, b_h, W_o, b_o):
    n_mols = a_scope.shape[0]
    sizes = a_scope[:, 1]
    seg_ids = jnp.repeat(jnp.arange(n_mols, dtype=jnp.int32), sizes,
                         total_repeat_length=N_ATOMS)
    pool = (seg_ids[:, None] == jnp.arange(n_mols, dtype=jnp.int32)[None, :])
    pool = pool.astype(jnp.float32) / sizes[None, :].astype(jnp.float32)

    n_pad = N_ATOMS_PAD - N_ATOMS
    pad_idx = (jnp.arange(n_pad * 16, dtype=jnp.int32) * 37) % N_BONDS
    a2b_flat = jnp.concatenate([a2b.reshape(-1), pad_idx])

    W_lo = W_h[:HALF].astype(jnp.bfloat16)
    W_hi = W_h[HALF:].astype(jnp.bfloat16)
    inp, M = _k0(f_bonds, W_i, b_i)
    for _ in range(DEPTH - 1):
        S = _sc_segsum(M, a2b_flat)
        D = _sc_diff(S, M, b2a, b2revb)
        M = _k3(D, inp, W_lo, W_hi, b_h)
    S = _sc_segsum(M, a2b_flat)
    return _k4(f_atoms, S, W_o, b_o, pool)


# speedup vs baseline: 1.8453x; 1.0740x over previous
import dataclasses
import functools

import jax
import jax.numpy as jnp
from jax import lax
from jax.experimental import pallas as pl
from jax.experimental.pallas import tpu as pltpu
from jax.experimental.pallas import tpu_sc as plsc

N_BONDS = 160000
N_ATOMS = 10000
HIDDEN = 1024
HALF = HIDDEN // 2
DEPTH = 8

NW = 32
N_ATOMS_PAD = 10240
APW = N_ATOMS_PAD // NW
BPW = N_BONDS // NW
SEG_C = 4
DIF_C = 16

BB = 2000
AB = 2000


def _sc_compiler_params():
    cp = pltpu.CompilerParams()
    if "needs_layout_passes" in pltpu.CompilerParams.__dataclass_fields__:
        cp = dataclasses.replace(cp, needs_layout_passes=False)
    return cp


_MASK_HI = -65536


def _tc_pack(x):
    lo = lax.bitcast_convert_type(x[:, :HALF], jnp.int32)
    hi = lax.bitcast_convert_type(x[:, HALF:], jnp.int32)
    lo_w = lax.shift_right_logical(lo + 32768, 16)
    hi_w = (hi + 32768) & _MASK_HI
    return lo_w | hi_w


def _tc_unpack_bf16(w):
    lo = lax.bitcast_convert_type(lax.shift_left(w, 16), jnp.float32)
    hi = lax.bitcast_convert_type(w & _MASK_HI, jnp.float32)
    return lo.astype(jnp.bfloat16), hi.astype(jnp.bfloat16)


def _sum16(rows_ref, stage_ref, n_atoms):
    @pl.loop(0, HALF, step=16)
    def _(c):
        for a in range(n_atoms):
            acc_a, acc_b = plsc.unpack(
                plsc.bitcast(rows_ref[a * 16, pl.ds(c, 16)], jnp.bfloat16),
                format=plsc.PackFormat.INTERLEAVED,
                preferred_element_type=jnp.float32)
            for i in range(1, 16):
                x_a, x_b = plsc.unpack(
                    plsc.bitcast(rows_ref[a * 16 + i, pl.ds(c, 16)],
                                 jnp.bfloat16),
                    format=plsc.PackFormat.INTERLEAVED,
                    preferred_element_type=jnp.float32)
                acc_a = acc_a + x_a
                acc_b = acc_b + x_b
            stage_ref[a, pl.ds(c, 16)] = plsc.bitcast(
                plsc.pack(acc_a, acc_b, format=plsc.PackFormat.INTERLEAVED),
                jnp.int32)


def _sc_segsum(M, a2b_flat):
    mesh = plsc.VectorSubcoreMesh(core_axis_name="c", subcore_axis_name="s")
    n_pairs = APW // SEG_C // 2

    @functools.partial(
        pl.kernel, mesh=mesh, compiler_params=_sc_compiler_params(),
        out_type=jax.ShapeDtypeStruct((N_ATOMS_PAD, HALF), jnp.int32),
        scratch_types=[
            pltpu.VMEM((APW * 16,), jnp.int32),
            pltpu.VMEM((SEG_C * 16, HALF), jnp.int32),
            pltpu.VMEM((SEG_C * 16, HALF), jnp.int32),
            pltpu.VMEM((SEG_C, HALF), jnp.int32),
            pltpu.VMEM((SEG_C, HALF), jnp.int32),
            pltpu.SemaphoreType.DMA,
            pltpu.SemaphoreType.DMA,
            pltpu.SemaphoreType.DMA,
            pltpu.SemaphoreType.DMA,
        ],
    )
    def k(m_hbm, idx_hbm, out_hbm, idx_v, r0, r1, st0, st1, g0, g1, s0, s1):
        wid = lax.axis_index("s") * 2 + lax.axis_index("c")
        pltpu.sync_copy(idx_hbm.at[pl.ds(wid * APW * 16, APW * 16)], idx_v)
        abase = wid * APW

        def gather(j, dst, sem):
            return pltpu.async_copy(
                m_hbm.at[idx_v.at[pl.ds(j * (SEG_C * 16), SEG_C * 16)]], dst, sem)

        def gwait(j, dst, sem):
            pltpu.make_async_copy(
                m_hbm.at[idx_v.at[pl.ds(j * (SEG_C * 16), SEG_C * 16)]], dst,
                sem).wait()

        def store(j, src, sem):
            return pltpu.async_copy(
                src, out_hbm.at[pl.ds(abase + j * SEG_C, SEG_C)], sem)

        def swait(src, sem):
            pltpu.make_async_copy(
                src, out_hbm.at[pl.ds(abase, SEG_C)], sem).wait()

        gather(0, r0, g0)
        gather(1, r1, g1)

        @pl.loop(0, n_pairs)
        def _(kk):
            j0 = 2 * kk
            j1 = j0 + 1

            @pl.when(kk > 0)
            def _():
                swait(st1, s1)
                gather(j1, r1, g1)

            gwait(j0, r0, g0)
            _sum16(r0, st0, SEG_C)
            store(j0, st0, s0)

            gwait(j1, r1, g1)
            _sum16(r1, st1, SEG_C)
            store(j1, st1, s1)

            @pl.when(kk < n_pairs - 1)
            def _():
                swait(st0, s0)
                gather(j0 + 2, r0, g0)

        swait(st0, s0)
        swait(st1, s1)

    return k(M, a2b_flat)


def _sc_diff(S, M, b2a, b2revb):
    mesh = plsc.VectorSubcoreMesh(core_axis_name="c", subcore_axis_name="s")
    n_full = BPW // DIF_C
    tail = BPW % DIF_C
    n_pairs = n_full // 2

    @functools.partial(
        pl.kernel, mesh=mesh, compiler_params=_sc_compiler_params(),
        out_type=jax.ShapeDtypeStruct((N_BONDS, HALF), jnp.int32),
        scratch_types=[
            pltpu.VMEM((BPW,), jnp.int32),
            pltpu.VMEM((BPW,), jnp.int32),
            pltpu.VMEM((DIF_C, HALF), jnp.int32),
            pltpu.VMEM((DIF_C, HALF), jnp.int32),
            pltpu.VMEM((DIF_C, HALF), jnp.int32),
            pltpu.VMEM((DIF_C, HALF), jnp.int32),
            pltpu.SemaphoreType.DMA,
            pltpu.SemaphoreType.DMA,
            pltpu.SemaphoreType.DMA,
            pltpu.SemaphoreType.DMA,
            pltpu.SemaphoreType.DMA,
            pltpu.SemaphoreType.DMA,
        ],
    )
    def k(s_hbm, m_hbm, ia_hbm, ir_hbm, out_hbm,
          ia_v, ir_v, sa0, sa1, mb0, mb1, ga0, ga1, gm0, gm1, ss0, ss1):
        wid = lax.axis_index("s") * 2 + lax.axis_index("c")
        pltpu.sync_copy(ia_hbm.at[pl.ds(wid * BPW, BPW)], ia_v)
        pltpu.sync_copy(ir_hbm.at[pl.ds(wid * BPW, BPW)], ir_v)
        bbase = wid * BPW

        def gathers(j, sdst, mdst, gsa, gsm):
            pltpu.async_copy(s_hbm.at[ia_v.at[pl.ds(j * DIF_C, DIF_C)]], sdst, gsa)
            pltpu.async_copy(m_hbm.at[ir_v.at[pl.ds(j * DIF_C, DIF_C)]], mdst, gsm)

        def gwaits(j, sdst, mdst, gsa, gsm):
            pltpu.make_async_copy(
                s_hbm.at[ia_v.at[pl.ds(j * DIF_C, DIF_C)]], sdst, gsa).wait()
            pltpu.make_async_copy(
                m_hbm.at[ir_v.at[pl.ds(j * DIF_C, DIF_C)]], mdst, gsm).wait()

        def sub_inplace(sdst, mdst, nrows=DIF_C):
            @pl.loop(0, HALF, step=16)
            def _(c):
                for r in range(nrows):
                    a = plsc.bitcast(sdst[r, pl.ds(c, 16)], jnp.bfloat16)
                    b = plsc.bitcast(mdst[r, pl.ds(c, 16)], jnp.bfloat16)
                    sdst[r, pl.ds(c, 16)] = plsc.bitcast(a - b, jnp.int32)

        def store(j, src, sem):
            pltpu.async_copy(src, out_hbm.at[pl.ds(bbase + j * DIF_C, DIF_C)], sem)

        def swait(src, sem):
            pltpu.make_async_copy(
                src, out_hbm.at[pl.ds(bbase, DIF_C)], sem).wait()

        gathers(0, sa0, mb0, ga0, gm0)
        gathers(1, sa1, mb1, ga1, gm1)

        @pl.loop(0, n_pairs)
        def _(kk):
            j0 = 2 * kk
            j1 = j0 + 1

            @pl.when(kk > 0)
            def _():
                swait(sa1, ss1)
                gathers(j1, sa1, mb1, ga1, gm1)

            gwaits(j0, sa0, mb0, ga0, gm0)
            sub_inplace(sa0, mb0)
            store(j0, sa0, ss0)

            gwaits(j1, sa1, mb1, ga1, gm1)
            sub_inplace(sa1, mb1)
            store(j1, sa1, ss1)

            swait(sa0, ss0)

            @pl.when(kk < n_pairs - 1)
            def _():
                gathers(j0 + 2, sa0, mb0, ga0, gm0)

        tbase = n_full * DIF_C
        sa0t = sa0.at[pl.ds(0, tail)]
        mb0t = mb0.at[pl.ds(0, tail)]
        pltpu.async_copy(s_hbm.at[ia_v.at[pl.ds(tbase, tail)]], sa0t, ga0)
        pltpu.async_copy(m_hbm.at[ir_v.at[pl.ds(tbase, tail)]], mb0t, gm0)
        pltpu.make_async_copy(
            s_hbm.at[ia_v.at[pl.ds(tbase, tail)]], sa0t, ga0).wait()
        pltpu.make_async_copy(
            m_hbm.at[ir_v.at[pl.ds(tbase, tail)]], mb0t, gm0).wait()
        sub_inplace(sa0, mb0, nrows=tail)
        pltpu.async_copy(sa0t, out_hbm.at[pl.ds(bbase + tbase, tail)], ss0)
        pltpu.make_async_copy(
            sa0t, out_hbm.at[pl.ds(bbase + tbase, tail)], ss0).wait()
        swait(sa1, ss1)

    return k(S, M, b2a, b2revb)


def _k0_body(fb_ref, wi_ref, bi_ref, inp_ref, m_ref):
    x = jnp.dot(fb_ref[...], wi_ref[...], preferred_element_type=jnp.float32)
    x = x + bi_ref[...]
    inp_ref[...] = x
    m_ref[...] = _tc_pack(jnp.maximum(x, 0.0))


def _k0(f_bonds, W_i, b_i):
    nfb = f_bonds.shape[1]
    grid = (N_BONDS // BB,)
    return pl.pallas_call(
        _k0_body,
        grid=grid,
        in_specs=[
            pl.BlockSpec((BB, nfb), lambda i: (i, 0)),
            pl.BlockSpec((nfb, HIDDEN), lambda i: (0, 0)),
            pl.BlockSpec((1, HIDDEN), lambda i: (0, 0)),
        ],
        out_specs=[
            pl.BlockSpec((BB, HIDDEN), lambda i: (i, 0)),
            pl.BlockSpec((BB, HALF), lambda i: (i, 0)),
        ],
        out_shape=[
            jax.ShapeDtypeStruct((N_BONDS, HIDDEN), jnp.float32),
            jax.ShapeDtypeStruct((N_BONDS, HALF), jnp.int32),
        ],
    )(f_bonds, W_i, b_i.reshape(1, HIDDEN))


def _k3_body(d_ref, inp_ref, wlo_ref, whi_ref, bh_ref, m_ref):
    lo, hi = _tc_unpack_bf16(d_ref[...])
    x = jnp.dot(lo, wlo_ref[...], preferred_element_type=jnp.float32)
    x += jnp.dot(hi, whi_ref[...], preferred_element_type=jnp.float32)
    m_ref[...] = _tc_pack(jnp.maximum(inp_ref[...] + x + bh_ref[...], 0.0))


def _k3(D, inp, W_lo, W_hi, b_h):
    grid = (N_BONDS // BB,)
    return pl.pallas_call(
        _k3_body,
        grid=grid,
        in_specs=[
            pl.BlockSpec((BB, HALF), lambda i: (i, 0)),
            pl.BlockSpec((BB, HIDDEN), lambda i: (i, 0)),
            pl.BlockSpec((HALF, HIDDEN), lambda i: (0, 0)),
            pl.BlockSpec((HALF, HIDDEN), lambda i: (0, 0)),
            pl.BlockSpec((1, HIDDEN), lambda i: (0, 0)),
        ],
        out_specs=pl.BlockSpec((BB, HALF), lambda i: (i, 0)),
        out_shape=jax.ShapeDtypeStruct((N_BONDS, HALF), jnp.int32),
    )(D, inp, W_lo, W_hi, b_h.reshape(1, HIDDEN))


def _k4_body(fa_ref, s_ref, woa_ref, wlo_ref, whi_ref, bo_ref, pool_ref,
             out_ref):
    h = jnp.dot(fa_ref[...], woa_ref[...], preferred_element_type=jnp.float32)
    lo, hi = _tc_unpack_bf16(s_ref[...])
    h += jnp.dot(lo, wlo_ref[...], preferred_element_type=jnp.float32)
    h += jnp.dot(hi, whi_ref[...], preferred_element_type=jnp.float32)
    h = jnp.maximum(h + bo_ref[...], 0.0)
    contrib = jax.lax.dot_general(
        pool_ref[...], h, (((0,), (0,)), ((), ())),
        preferred_element_type=jnp.float32)

    @pl.when(pl.program_id(0) == 0)
    def _():
        out_ref[...] = jnp.zeros_like(out_ref)

    out_ref[...] += contrib


def _k4(f_atoms, S, W_o, b_o, pool):
    nfa = f_atoms.shape[1]
    n_mols = pool.shape[1]
    W_oa = W_o[:nfa]
    W_lo = W_o[nfa:nfa + HALF].astype(jnp.bfloat16)
    W_hi = W_o[nfa + HALF:].astype(jnp.bfloat16)
    grid = (N_ATOMS // AB,)
    return pl.pallas_call(
        _k4_body,
        grid=grid,
        in_specs=[
            pl.BlockSpec((AB, nfa), lambda i: (i, 0)),
            pl.BlockSpec((AB, HALF), lambda i: (i, 0)),
            pl.BlockSpec((nfa, HIDDEN), lambda i: (0, 0)),
            pl.BlockSpec((HALF, HIDDEN), lambda i: (0, 0)),
            pl.BlockSpec((HALF, HIDDEN), lambda i: (0, 0)),
            pl.BlockSpec((1, HIDDEN), lambda i: (0, 0)),
            pl.BlockSpec((AB, n_mols), lambda i: (i, 0)),
        ],
        out_specs=pl.BlockSpec((n_mols, HIDDEN), lambda i: (0, 0)),
        out_shape=jax.ShapeDtypeStruct((n_mols, HIDDEN), jnp.float32),
    )(f_atoms, S, W_oa, W_lo, W_hi, b_o.reshape(1, HIDDEN), pool)


def kernel(f_atoms, f_bonds, a2b, b2a, b2revb, a_scope, W_i, b_i, W_h, b_h, W_o, b_o):
    n_mols = a_scope.shape[0]
    sizes = a_scope[:, 1]
    seg_ids = jnp.repeat(jnp.arange(n_mols, dtype=jnp.int32), sizes,
                         total_repeat_length=N_ATOMS)
    pool = (seg_ids[:, None] == jnp.arange(n_mols, dtype=jnp.int32)[None, :])
    pool = pool.astype(jnp.float32) / sizes[None, :].astype(jnp.float32)

    n_pad = N_ATOMS_PAD - N_ATOMS
    pad_idx = (jnp.arange(n_pad * 16, dtype=jnp.int32) * 37) % N_BONDS
    a2b_flat = jnp.concatenate([a2b.reshape(-1), pad_idx])

    W_lo = W_h[:HALF].astype(jnp.bfloat16)
    W_hi = W_h[HALF:].astype(jnp.bfloat16)
    inp, M = _k0(f_bonds, W_i, b_i)
    for _ in range(DEPTH - 1):
        S = _sc_segsum(M, a2b_flat)
        D = _sc_diff(S, M, b2a, b2revb)
        M = _k3(D, inp, W_lo, W_hi, b_h)
    S = _sc_segsum(M, a2b_flat)
    return _k4(f_atoms, S, W_o, b_o, pool)


# speedup vs baseline: 1.8778x; 1.0176x over previous
import dataclasses
import functools

import jax
import jax.numpy as jnp
from jax import lax
from jax.experimental import pallas as pl
from jax.experimental.pallas import tpu as pltpu
from jax.experimental.pallas import tpu_sc as plsc

N_BONDS = 160000
N_ATOMS = 10000
HIDDEN = 1024
HALF = HIDDEN // 2
DEPTH = 8

NW = 32
N_ATOMS_PAD = 10240
APW = N_ATOMS_PAD // NW
BPW = N_BONDS // NW
SEG_C = 4
DIF_C = 16

BB = 2000
AB = 2000


def _sc_compiler_params():
    cp = pltpu.CompilerParams()
    if "needs_layout_passes" in pltpu.CompilerParams.__dataclass_fields__:
        cp = dataclasses.replace(cp, needs_layout_passes=False)
    return cp


_MASK_HI = -65536


def _tc_pack_halves(lo_f, hi_f):
    lo = lax.bitcast_convert_type(lo_f, jnp.int32)
    hi = lax.bitcast_convert_type(hi_f, jnp.int32)
    lo_w = lax.shift_right_logical(lo + 32768, 16)
    hi_w = (hi + 32768) & _MASK_HI
    return lo_w | hi_w


def _tc_pack(x):
    return _tc_pack_halves(x[:, :HALF], x[:, HALF:])


def _tc_unpack_f32(w):
    lo = lax.bitcast_convert_type(lax.shift_left(w, 16), jnp.float32)
    hi = lax.bitcast_convert_type(w & _MASK_HI, jnp.float32)
    return lo, hi


def _tc_unpack_bf16(w):
    lo, hi = _tc_unpack_f32(w)
    return lo.astype(jnp.bfloat16), hi.astype(jnp.bfloat16)


def _sum16(rows_ref, stage_ref, n_atoms):
    @pl.loop(0, HALF, step=16)
    def _(c):
        for a in range(n_atoms):
            acc_a, acc_b = plsc.unpack(
                plsc.bitcast(rows_ref[a * 16, pl.ds(c, 16)], jnp.bfloat16),
                format=plsc.PackFormat.INTERLEAVED,
                preferred_element_type=jnp.float32)
            for i in range(1, 16):
                x_a, x_b = plsc.unpack(
                    plsc.bitcast(rows_ref[a * 16 + i, pl.ds(c, 16)],
                                 jnp.bfloat16),
                    format=plsc.PackFormat.INTERLEAVED,
                    preferred_element_type=jnp.float32)
                acc_a = acc_a + x_a
                acc_b = acc_b + x_b
            stage_ref[a, pl.ds(c, 16)] = plsc.bitcast(
                plsc.pack(acc_a, acc_b, format=plsc.PackFormat.INTERLEAVED),
                jnp.int32)


def _sc_segsum(M, a2b_flat):
    mesh = plsc.VectorSubcoreMesh(core_axis_name="c", subcore_axis_name="s")
    n_pairs = APW // SEG_C // 2

    @functools.partial(
        pl.kernel, mesh=mesh, compiler_params=_sc_compiler_params(),
        out_type=jax.ShapeDtypeStruct((N_ATOMS_PAD, HALF), jnp.int32),
        scratch_types=[
            pltpu.VMEM((APW * 16,), jnp.int32),
            pltpu.VMEM((SEG_C * 16, HALF), jnp.int32),
            pltpu.VMEM((SEG_C * 16, HALF), jnp.int32),
            pltpu.VMEM((SEG_C, HALF), jnp.int32),
            pltpu.VMEM((SEG_C, HALF), jnp.int32),
            pltpu.SemaphoreType.DMA,
            pltpu.SemaphoreType.DMA,
            pltpu.SemaphoreType.DMA,
            pltpu.SemaphoreType.DMA,
        ],
    )
    def k(m_hbm, idx_hbm, out_hbm, idx_v, r0, r1, st0, st1, g0, g1, s0, s1):
        wid = lax.axis_index("s") * 2 + lax.axis_index("c")
        pltpu.sync_copy(idx_hbm.at[pl.ds(wid * APW * 16, APW * 16)], idx_v)
        abase = wid * APW

        def gather(j, dst, sem):
            return pltpu.async_copy(
                m_hbm.at[idx_v.at[pl.ds(j * (SEG_C * 16), SEG_C * 16)]], dst, sem)

        def gwait(j, dst, sem):
            pltpu.make_async_copy(
                m_hbm.at[idx_v.at[pl.ds(j * (SEG_C * 16), SEG_C * 16)]], dst,
                sem).wait()

        def store(j, src, sem):
            return pltpu.async_copy(
                src, out_hbm.at[pl.ds(abase + j * SEG_C, SEG_C)], sem)

        def swait(src, sem):
            pltpu.make_async_copy(
                src, out_hbm.at[pl.ds(abase, SEG_C)], sem).wait()

        gather(0, r0, g0)
        gather(1, r1, g1)

        @pl.loop(0, n_pairs)
        def _(kk):
            j0 = 2 * kk
            j1 = j0 + 1

            @pl.when(kk > 0)
            def _():
                swait(st1, s1)
                gather(j1, r1, g1)

            gwait(j0, r0, g0)
            _sum16(r0, st0, SEG_C)
            store(j0, st0, s0)

            gwait(j1, r1, g1)
            _sum16(r1, st1, SEG_C)
            store(j1, st1, s1)

            @pl.when(kk < n_pairs - 1)
            def _():
                swait(st0, s0)
                gather(j0 + 2, r0, g0)

        swait(st0, s0)
        swait(st1, s1)

    return k(M, a2b_flat)


def _sc_diff(S, M, b2a, b2revb):
    mesh = plsc.VectorSubcoreMesh(core_axis_name="c", subcore_axis_name="s")
    n_full = BPW // DIF_C
    tail = BPW % DIF_C
    n_pairs = n_full // 2

    @functools.partial(
        pl.kernel, mesh=mesh, compiler_params=_sc_compiler_params(),
        out_type=jax.ShapeDtypeStruct((N_BONDS, HALF), jnp.int32),
        scratch_types=[
            pltpu.VMEM((BPW,), jnp.int32),
            pltpu.VMEM((BPW,), jnp.int32),
            pltpu.VMEM((DIF_C, HALF), jnp.int32),
            pltpu.VMEM((DIF_C, HALF), jnp.int32),
            pltpu.VMEM((DIF_C, HALF), jnp.int32),
            pltpu.VMEM((DIF_C, HALF), jnp.int32),
            pltpu.SemaphoreType.DMA,
            pltpu.SemaphoreType.DMA,
            pltpu.SemaphoreType.DMA,
            pltpu.SemaphoreType.DMA,
            pltpu.SemaphoreType.DMA,
            pltpu.SemaphoreType.DMA,
        ],
    )
    def k(s_hbm, m_hbm, ia_hbm, ir_hbm, out_hbm,
          ia_v, ir_v, sa0, sa1, mb0, mb1, ga0, ga1, gm0, gm1, ss0, ss1):
        wid = lax.axis_index("s") * 2 + lax.axis_index("c")
        pltpu.sync_copy(ia_hbm.at[pl.ds(wid * BPW, BPW)], ia_v)
        pltpu.sync_copy(ir_hbm.at[pl.ds(wid * BPW, BPW)], ir_v)
        bbase = wid * BPW

        def gathers(j, sdst, mdst, gsa, gsm):
            pltpu.async_copy(s_hbm.at[ia_v.at[pl.ds(j * DIF_C, DIF_C)]], sdst, gsa)
            pltpu.async_copy(m_hbm.at[ir_v.at[pl.ds(j * DIF_C, DIF_C)]], mdst, gsm)

        def gwaits(j, sdst, mdst, gsa, gsm):
            pltpu.make_async_copy(
                s_hbm.at[ia_v.at[pl.ds(j * DIF_C, DIF_C)]], sdst, gsa).wait()
            pltpu.make_async_copy(
                m_hbm.at[ir_v.at[pl.ds(j * DIF_C, DIF_C)]], mdst, gsm).wait()

        def sub_inplace(sdst, mdst, nrows=DIF_C):
            @pl.loop(0, HALF, step=16)
            def _(c):
                for r in range(nrows):
                    a = plsc.bitcast(sdst[r, pl.ds(c, 16)], jnp.bfloat16)
                    b = plsc.bitcast(mdst[r, pl.ds(c, 16)], jnp.bfloat16)
                    sdst[r, pl.ds(c, 16)] = plsc.bitcast(a - b, jnp.int32)

        def store(j, src, sem):
            pltpu.async_copy(src, out_hbm.at[pl.ds(bbase + j * DIF_C, DIF_C)], sem)

        def swait(src, sem):
            pltpu.make_async_copy(
                src, out_hbm.at[pl.ds(bbase, DIF_C)], sem).wait()

        gathers(0, sa0, mb0, ga0, gm0)
        gathers(1, sa1, mb1, ga1, gm1)

        @pl.loop(0, n_pairs)
        def _(kk):
            j0 = 2 * kk
            j1 = j0 + 1

            @pl.when(kk > 0)
            def _():
                swait(sa1, ss1)
                gathers(j1, sa1, mb1, ga1, gm1)

            gwaits(j0, sa0, mb0, ga0, gm0)
            sub_inplace(sa0, mb0)
            store(j0, sa0, ss0)

            gwaits(j1, sa1, mb1, ga1, gm1)
            sub_inplace(sa1, mb1)
            store(j1, sa1, ss1)

            swait(sa0, ss0)

            @pl.when(kk < n_pairs - 1)
            def _():
                gathers(j0 + 2, sa0, mb0, ga0, gm0)

        tbase = n_full * DIF_C
        sa0t = sa0.at[pl.ds(0, tail)]
        mb0t = mb0.at[pl.ds(0, tail)]
        pltpu.async_copy(s_hbm.at[ia_v.at[pl.ds(tbase, tail)]], sa0t, ga0)
        pltpu.async_copy(m_hbm.at[ir_v.at[pl.ds(tbase, tail)]], mb0t, gm0)
        pltpu.make_async_copy(
            s_hbm.at[ia_v.at[pl.ds(tbase, tail)]], sa0t, ga0).wait()
        pltpu.make_async_copy(
            m_hbm.at[ir_v.at[pl.ds(tbase, tail)]], mb0t, gm0).wait()
        sub_inplace(sa0, mb0, nrows=tail)
        pltpu.async_copy(sa0t, out_hbm.at[pl.ds(bbase + tbase, tail)], ss0)
        pltpu.make_async_copy(
            sa0t, out_hbm.at[pl.ds(bbase + tbase, tail)], ss0).wait()
        swait(sa1, ss1)

    return k(S, M, b2a, b2revb)


def _k0_body(fb_ref, wi_ref, bi_ref, inp_ref, m_ref):
    x = jnp.dot(fb_ref[...], wi_ref[...], preferred_element_type=jnp.float32)
    x = x + bi_ref[...]
    inp_ref[...] = _tc_pack(x)
    m_ref[...] = _tc_pack(jnp.maximum(x, 0.0))


def _k0(f_bonds, W_i, b_i):
    nfb = f_bonds.shape[1]
    grid = (N_BONDS // BB,)
    return pl.pallas_call(
        _k0_body,
        grid=grid,
        in_specs=[
            pl.BlockSpec((BB, nfb), lambda i: (i, 0)),
            pl.BlockSpec((nfb, HIDDEN), lambda i: (0, 0)),
            pl.BlockSpec((1, HIDDEN), lambda i: (0, 0)),
        ],
        out_specs=[
            pl.BlockSpec((BB, HALF), lambda i: (i, 0)),
            pl.BlockSpec((BB, HALF), lambda i: (i, 0)),
        ],
        out_shape=[
            jax.ShapeDtypeStruct((N_BONDS, HALF), jnp.int32),
            jax.ShapeDtypeStruct((N_BONDS, HALF), jnp.int32),
        ],
    )(f_bonds, W_i, b_i.reshape(1, HIDDEN))


def _k3_body(d_ref, inp_ref, wlo_ref, whi_ref, bh_ref, m_ref):
    lo, hi = _tc_unpack_bf16(d_ref[...])
    x = jnp.dot(lo, wlo_ref[...], preferred_element_type=jnp.float32)
    x += jnp.dot(hi, whi_ref[...], preferred_element_type=jnp.float32)
    ilo, ihi = _tc_unpack_f32(inp_ref[...])
    b = bh_ref[...]
    m_lo = jnp.maximum(x[:, :HALF] + ilo + b[:, :HALF], 0.0)
    m_hi = jnp.maximum(x[:, HALF:] + ihi + b[:, HALF:], 0.0)
    m_ref[...] = _tc_pack_halves(m_lo, m_hi)


def _k3(D, inp, W_lo, W_hi, b_h):
    grid = (N_BONDS // BB,)
    return pl.pallas_call(
        _k3_body,
        grid=grid,
        in_specs=[
            pl.BlockSpec((BB, HALF), lambda i: (i, 0)),
            pl.BlockSpec((BB, HALF), lambda i: (i, 0)),
            pl.BlockSpec((HALF, HIDDEN), lambda i: (0, 0)),
            pl.BlockSpec((HALF, HIDDEN), lambda i: (0, 0)),
            pl.BlockSpec((1, HIDDEN), lambda i: (0, 0)),
        ],
        out_specs=pl.BlockSpec((BB, HALF), lambda i: (i, 0)),
        out_shape=jax.ShapeDtypeStruct((N_BONDS, HALF), jnp.int32),
    )(D, inp, W_lo, W_hi, b_h.reshape(1, HIDDEN))


def _k4_body(fa_ref, s_ref, woa_ref, wlo_ref, whi_ref, bo_ref, pool_ref,
             out_ref):
    h = jnp.dot(fa_ref[...], woa_ref[...], preferred_element_type=jnp.float32)
    lo, hi = _tc_unpack_bf16(s_ref[...])
    h += jnp.dot(lo, wlo_ref[...], preferred_element_type=jnp.float32)
    h += jnp.dot(hi, whi_ref[...], preferred_element_type=jnp.float32)
    h = jnp.maximum(h + bo_ref[...], 0.0)
    contrib = jax.lax.dot_general(
        pool_ref[...], h, (((0,), (0,)), ((), ())),
        preferred_element_type=jnp.float32)

    @pl.when(pl.program_id(0) == 0)
    def _():
        out_ref[...] = jnp.zeros_like(out_ref)

    out_ref[...] += contrib


def _k4(f_atoms, S, W_o, b_o, pool):
    nfa = f_atoms.shape[1]
    n_mols = pool.shape[1]
    W_oa = W_o[:nfa]
    W_lo = W_o[nfa:nfa + HALF].astype(jnp.bfloat16)
    W_hi = W_o[nfa + HALF:].astype(jnp.bfloat16)
    grid = (N_ATOMS // AB,)
    return pl.pallas_call(
        _k4_body,
        grid=grid,
        in_specs=[
            pl.BlockSpec((AB, nfa), lambda i: (i, 0)),
            pl.BlockSpec((AB, HALF), lambda i: (i, 0)),
            pl.BlockSpec((nfa, HIDDEN), lambda i: (0, 0)),
            pl.BlockSpec((HALF, HIDDEN), lambda i: (0, 0)),
            pl.BlockSpec((HALF, HIDDEN), lambda i: (0, 0)),
            pl.BlockSpec((1, HIDDEN), lambda i: (0, 0)),
            pl.BlockSpec((AB, n_mols), lambda i: (i, 0)),
        ],
        out_specs=pl.BlockSpec((n_mols, HIDDEN), lambda i: (0, 0)),
        out_shape=jax.ShapeDtypeStruct((n_mols, HIDDEN), jnp.float32),
    )(f_atoms, S, W_oa, W_lo, W_hi, b_o.reshape(1, HIDDEN), pool)


def kernel(f_atoms, f_bonds, a2b, b2a, b2revb, a_scope, W_i, b_i, W_h, b_h, W_o, b_o):
    n_mols = a_scope.shape[0]
    sizes = a_scope[:, 1]
    seg_ids = jnp.repeat(jnp.arange(n_mols, dtype=jnp.int32), sizes,
                         total_repeat_length=N_ATOMS)
    pool = (seg_ids[:, None] == jnp.arange(n_mols, dtype=jnp.int32)[None, :])
    pool = pool.astype(jnp.float32) / sizes[None, :].astype(jnp.float32)

    n_pad = N_ATOMS_PAD - N_ATOMS
    pad_idx = (jnp.arange(n_pad * 16, dtype=jnp.int32) * 37) % N_BONDS
    a2b_flat = jnp.concatenate([a2b.reshape(-1), pad_idx])

    W_lo = W_h[:HALF].astype(jnp.bfloat16)
    W_hi = W_h[HALF:].astype(jnp.bfloat16)
    inp, M = _k0(f_bonds, W_i, b_i)
    for _ in range(DEPTH - 1):
        S = _sc_segsum(M, a2b_flat)
        D = _sc_diff(S, M, b2a, b2revb)
        M = _k3(D, inp, W_lo, W_hi, b_h)
    S = _sc_segsum(M, a2b_flat)
    return _k4(f_atoms, S, W_o, b_o, pool)


# speedup vs baseline: 2.0475x; 1.0904x over previous
import dataclasses
import functools

import jax
import jax.numpy as jnp
from jax import lax
from jax.experimental import pallas as pl
from jax.experimental.pallas import tpu as pltpu
from jax.experimental.pallas import tpu_sc as plsc

N_BONDS = 160000
N_ATOMS = 10000
HIDDEN = 1024
HALF = HIDDEN // 2
DEPTH = 8

NW = 32
N_ATOMS_PAD = 10240
APW = N_ATOMS_PAD // NW
BPW = N_BONDS // NW
SEG_C = 4
DIF_C = 16

BB = 2000
AB = 2000


def _sc_compiler_params():
    cp = pltpu.CompilerParams()
    if "needs_layout_passes" in pltpu.CompilerParams.__dataclass_fields__:
        cp = dataclasses.replace(cp, needs_layout_passes=False)
    return cp


_MASK_HI = -65536


def _tc_pack_halves(lo_f, hi_f):
    lo = lax.bitcast_convert_type(lo_f, jnp.int32)
    hi = lax.bitcast_convert_type(hi_f, jnp.int32)
    lo_w = lax.shift_right_logical(lo + 32768, 16)
    hi_w = (hi + 32768) & _MASK_HI
    return lo_w | hi_w


def _tc_pack(x):
    return _tc_pack_halves(x[:, :HALF], x[:, HALF:])


def _tc_unpack_f32(w):
    lo = lax.bitcast_convert_type(lax.shift_left(w, 16), jnp.float32)
    hi = lax.bitcast_convert_type(w & _MASK_HI, jnp.float32)
    return lo, hi


def _tc_unpack_bf16(w):
    lo, hi = _tc_unpack_f32(w)
    return lo.astype(jnp.bfloat16), hi.astype(jnp.bfloat16)


def _sum16(rows_ref, stage_ref, n_atoms):
    @pl.loop(0, HALF, step=16)
    def _(c):
        for a in range(n_atoms):
            acc_a, acc_b = plsc.unpack(
                plsc.bitcast(rows_ref[a * 16, pl.ds(c, 16)], jnp.bfloat16),
                format=plsc.PackFormat.INTERLEAVED,
                preferred_element_type=jnp.float32)
            for i in range(1, 16):
                x_a, x_b = plsc.unpack(
                    plsc.bitcast(rows_ref[a * 16 + i, pl.ds(c, 16)],
                                 jnp.bfloat16),
                    format=plsc.PackFormat.INTERLEAVED,
                    preferred_element_type=jnp.float32)
                acc_a = acc_a + x_a
                acc_b = acc_b + x_b
            stage_ref[a, pl.ds(c, 16)] = plsc.bitcast(
                plsc.pack(acc_a, acc_b, format=plsc.PackFormat.INTERLEAVED),
                jnp.int32)


def _sc_segsum(M, a2b_flat):
    mesh = plsc.VectorSubcoreMesh(core_axis_name="c", subcore_axis_name="s")
    n_pairs = APW // SEG_C // 2

    @functools.partial(
        pl.kernel, mesh=mesh, compiler_params=_sc_compiler_params(),
        out_type=jax.ShapeDtypeStruct((N_ATOMS_PAD, HALF), jnp.int32),
        scratch_types=[
            pltpu.VMEM((APW * 16,), jnp.int32),
            pltpu.VMEM((SEG_C * 16, HALF), jnp.int32),
            pltpu.VMEM((SEG_C * 16, HALF), jnp.int32),
            pltpu.VMEM((SEG_C, HALF), jnp.int32),
            pltpu.VMEM((SEG_C, HALF), jnp.int32),
            pltpu.SemaphoreType.DMA,
            pltpu.SemaphoreType.DMA,
            pltpu.SemaphoreType.DMA,
            pltpu.SemaphoreType.DMA,
        ],
    )
    def k(m_hbm, idx_hbm, out_hbm, idx_v, r0, r1, st0, st1, g0, g1, s0, s1):
        wid = lax.axis_index("s") * 2 + lax.axis_index("c")
        pltpu.sync_copy(idx_hbm.at[pl.ds(wid * APW * 16, APW * 16)], idx_v)
        abase = wid * APW

        def gather(j, dst, sem):
            return pltpu.async_copy(
                m_hbm.at[idx_v.at[pl.ds(j * (SEG_C * 16), SEG_C * 16)]], dst, sem)

        def gwait(j, dst, sem):
            pltpu.make_async_copy(
                m_hbm.at[idx_v.at[pl.ds(j * (SEG_C * 16), SEG_C * 16)]], dst,
                sem).wait()

        def store(j, src, sem):
            return pltpu.async_copy(
                src, out_hbm.at[pl.ds(abase + j * SEG_C, SEG_C)], sem)

        def swait(src, sem):
            pltpu.make_async_copy(
                src, out_hbm.at[pl.ds(abase, SEG_C)], sem).wait()

        gather(0, r0, g0)
        gather(1, r1, g1)

        @pl.loop(0, n_pairs)
        def _(kk):
            j0 = 2 * kk
            j1 = j0 + 1

            @pl.when(kk > 0)
            def _():
                swait(st1, s1)
                gather(j1, r1, g1)

            gwait(j0, r0, g0)
            _sum16(r0, st0, SEG_C)
            store(j0, st0, s0)

            gwait(j1, r1, g1)
            _sum16(r1, st1, SEG_C)
            store(j1, st1, s1)

            @pl.when(kk < n_pairs - 1)
            def _():
                swait(st0, s0)
                gather(j0 + 2, r0, g0)

        swait(st0, s0)
        swait(st1, s1)

    return k(M, a2b_flat)


def _sc_diff(S, M, b2a, b2revb):
    mesh = plsc.VectorSubcoreMesh(core_axis_name="c", subcore_axis_name="s")
    nb = b2a.shape[0]
    bpw = nb // NW
    n_full = bpw // DIF_C
    tail = bpw % DIF_C
    n_pairs = n_full // 2
    assert bpw % 8 == 0 and n_full % 2 == 0 and 0 < tail < DIF_C, (nb, bpw)

    @functools.partial(
        pl.kernel, mesh=mesh, compiler_params=_sc_compiler_params(),
        out_type=jax.ShapeDtypeStruct((nb, HALF), jnp.int32),
        scratch_types=[
            pltpu.VMEM((bpw,), jnp.int32),
            pltpu.VMEM((bpw,), jnp.int32),
            pltpu.VMEM((DIF_C, HALF), jnp.int32),
            pltpu.VMEM((DIF_C, HALF), jnp.int32),
            pltpu.VMEM((DIF_C, HALF), jnp.int32),
            pltpu.VMEM((DIF_C, HALF), jnp.int32),
            pltpu.SemaphoreType.DMA,
            pltpu.SemaphoreType.DMA,
            pltpu.SemaphoreType.DMA,
            pltpu.SemaphoreType.DMA,
            pltpu.SemaphoreType.DMA,
            pltpu.SemaphoreType.DMA,
        ],
    )
    def k(s_hbm, m_hbm, ia_hbm, ir_hbm, out_hbm,
          ia_v, ir_v, sa0, sa1, mb0, mb1, ga0, ga1, gm0, gm1, ss0, ss1):
        wid = lax.axis_index("s") * 2 + lax.axis_index("c")
        pltpu.sync_copy(ia_hbm.at[pl.ds(wid * bpw, bpw)], ia_v)
        pltpu.sync_copy(ir_hbm.at[pl.ds(wid * bpw, bpw)], ir_v)
        bbase = wid * bpw

        def gathers(j, sdst, mdst, gsa, gsm):
            pltpu.async_copy(s_hbm.at[ia_v.at[pl.ds(j * DIF_C, DIF_C)]], sdst, gsa)
            pltpu.async_copy(m_hbm.at[ir_v.at[pl.ds(j * DIF_C, DIF_C)]], mdst, gsm)

        def gwaits(j, sdst, mdst, gsa, gsm):
            pltpu.make_async_copy(
                s_hbm.at[ia_v.at[pl.ds(j * DIF_C, DIF_C)]], sdst, gsa).wait()
            pltpu.make_async_copy(
                m_hbm.at[ir_v.at[pl.ds(j * DIF_C, DIF_C)]], mdst, gsm).wait()

        def sub_inplace(sdst, mdst, nrows=DIF_C):
            @pl.loop(0, HALF, step=16)
            def _(c):
                for r in range(nrows):
                    a = plsc.bitcast(sdst[r, pl.ds(c, 16)], jnp.bfloat16)
                    b = plsc.bitcast(mdst[r, pl.ds(c, 16)], jnp.bfloat16)
                    sdst[r, pl.ds(c, 16)] = plsc.bitcast(a - b, jnp.int32)

        def store(j, src, sem):
            pltpu.async_copy(src, out_hbm.at[pl.ds(bbase + j * DIF_C, DIF_C)], sem)

        def swait(src, sem):
            pltpu.make_async_copy(
                src, out_hbm.at[pl.ds(bbase, DIF_C)], sem).wait()

        gathers(0, sa0, mb0, ga0, gm0)
        gathers(1, sa1, mb1, ga1, gm1)

        @pl.loop(0, n_pairs)
        def _(kk):
            j0 = 2 * kk
            j1 = j0 + 1

            @pl.when(kk > 0)
            def _():
                swait(sa1, ss1)
                gathers(j1, sa1, mb1, ga1, gm1)

            gwaits(j0, sa0, mb0, ga0, gm0)
            sub_inplace(sa0, mb0)
            store(j0, sa0, ss0)

            gwaits(j1, sa1, mb1, ga1, gm1)
            sub_inplace(sa1, mb1)
            store(j1, sa1, ss1)

            swait(sa0, ss0)

            @pl.when(kk < n_pairs - 1)
            def _():
                gathers(j0 + 2, sa0, mb0, ga0, gm0)

        tbase = n_full * DIF_C
        sa0t = sa0.at[pl.ds(0, tail)]
        mb0t = mb0.at[pl.ds(0, tail)]
        pltpu.async_copy(s_hbm.at[ia_v.at[pl.ds(tbase, tail)]], sa0t, ga0)
        pltpu.async_copy(m_hbm.at[ir_v.at[pl.ds(tbase, tail)]], mb0t, gm0)
        pltpu.make_async_copy(
            s_hbm.at[ia_v.at[pl.ds(tbase, tail)]], sa0t, ga0).wait()
        pltpu.make_async_copy(
            m_hbm.at[ir_v.at[pl.ds(tbase, tail)]], mb0t, gm0).wait()
        sub_inplace(sa0, mb0, nrows=tail)
        pltpu.async_copy(sa0t, out_hbm.at[pl.ds(bbase + tbase, tail)], ss0)
        pltpu.make_async_copy(
            sa0t, out_hbm.at[pl.ds(bbase + tbase, tail)], ss0).wait()
        swait(sa1, ss1)

    return k(S, M, b2a, b2revb)


def _k0_body(fb_ref, wi_ref, bi_ref, inp_ref, m_ref):
    x = jnp.dot(fb_ref[...], wi_ref[...], preferred_element_type=jnp.float32)
    x = x + bi_ref[...]
    inp_ref[...] = _tc_pack(x)
    m_ref[...] = _tc_pack(jnp.maximum(x, 0.0))


def _k0(f_bonds, W_i, b_i):
    nfb = f_bonds.shape[1]
    grid = (N_BONDS // BB,)
    return pl.pallas_call(
        _k0_body,
        grid=grid,
        in_specs=[
            pl.BlockSpec((BB, nfb), lambda i: (i, 0)),
            pl.BlockSpec((nfb, HIDDEN), lambda i: (0, 0)),
            pl.BlockSpec((1, HIDDEN), lambda i: (0, 0)),
        ],
        out_specs=[
            pl.BlockSpec((BB, HALF), lambda i: (i, 0)),
            pl.BlockSpec((BB, HALF), lambda i: (i, 0)),
        ],
        out_shape=[
            jax.ShapeDtypeStruct((N_BONDS, HALF), jnp.int32),
            jax.ShapeDtypeStruct((N_BONDS, HALF), jnp.int32),
        ],
    )(f_bonds, W_i, b_i.reshape(1, HIDDEN))


def _k3_body(d_ref, inp_ref, wlo_ref, whi_ref, bh_ref, m_ref):
    lo, hi = _tc_unpack_bf16(d_ref[...])
    x = jnp.dot(lo, wlo_ref[...], preferred_element_type=jnp.float32)
    x += jnp.dot(hi, whi_ref[...], preferred_element_type=jnp.float32)
    ilo, ihi = _tc_unpack_f32(inp_ref[...])
    b = bh_ref[...]
    m_lo = jnp.maximum(x[:, :HALF] + ilo + b[:, :HALF], 0.0)
    m_hi = jnp.maximum(x[:, HALF:] + ihi + b[:, HALF:], 0.0)
    m_ref[...] = _tc_pack_halves(m_lo, m_hi)


def _k3_body_dead(d_ref, inp_ref, wlo_ref, whi_ref, bh_ref, dead_ref, m_ref):
    _k3_body(d_ref, inp_ref, wlo_ref, whi_ref, bh_ref, m_ref)


def _k3r(D, inp, dead, W_lo, W_hi, b_h, half):
    NH = N_BONDS // 2
    off = half * (NH // BB)
    grid = (NH // BB,)
    return pl.pallas_call(
        _k3_body_dead,
        grid=grid,
        in_specs=[
            pl.BlockSpec((BB, HALF), lambda i: (i, 0)),
            pl.BlockSpec((BB, HALF), lambda i: (i + off, 0)),
            pl.BlockSpec((HALF, HIDDEN), lambda i: (0, 0)),
            pl.BlockSpec((HALF, HIDDEN), lambda i: (0, 0)),
            pl.BlockSpec((1, HIDDEN), lambda i: (0, 0)),
            pl.BlockSpec(memory_space=pl.ANY),
        ],
        out_specs=pl.BlockSpec((BB, HALF), lambda i: (i + off, 0)),
        out_shape=jax.ShapeDtypeStruct((N_BONDS, HALF), jnp.int32),
        input_output_aliases={5: 0},
    )(D, inp, W_lo, W_hi, b_h.reshape(1, HIDDEN), dead)


def _k4_body(fa_ref, s_ref, woa_ref, wlo_ref, whi_ref, bo_ref, pool_ref,
             out_ref):
    h = jnp.dot(fa_ref[...], woa_ref[...], preferred_element_type=jnp.float32)
    lo, hi = _tc_unpack_bf16(s_ref[...])
    h += jnp.dot(lo, wlo_ref[...], preferred_element_type=jnp.float32)
    h += jnp.dot(hi, whi_ref[...], preferred_element_type=jnp.float32)
    h = jnp.maximum(h + bo_ref[...], 0.0)
    contrib = jax.lax.dot_general(
        pool_ref[...], h, (((0,), (0,)), ((), ())),
        preferred_element_type=jnp.float32)

    @pl.when(pl.program_id(0) == 0)
    def _():
        out_ref[...] = jnp.zeros_like(out_ref)

    out_ref[...] += contrib


def _k4(f_atoms, S, W_o, b_o, pool):
    nfa = f_atoms.shape[1]
    n_mols = pool.shape[1]
    W_oa = W_o[:nfa]
    W_lo = W_o[nfa:nfa + HALF].astype(jnp.bfloat16)
    W_hi = W_o[nfa + HALF:].astype(jnp.bfloat16)
    grid = (N_ATOMS // AB,)
    return pl.pallas_call(
        _k4_body,
        grid=grid,
        in_specs=[
            pl.BlockSpec((AB, nfa), lambda i: (i, 0)),
            pl.BlockSpec((AB, HALF), lambda i: (i, 0)),
            pl.BlockSpec((nfa, HIDDEN), lambda i: (0, 0)),
            pl.BlockSpec((HALF, HIDDEN), lambda i: (0, 0)),
            pl.BlockSpec((HALF, HIDDEN), lambda i: (0, 0)),
            pl.BlockSpec((1, HIDDEN), lambda i: (0, 0)),
            pl.BlockSpec((AB, n_mols), lambda i: (i, 0)),
        ],
        out_specs=pl.BlockSpec((n_mols, HIDDEN), lambda i: (0, 0)),
        out_shape=jax.ShapeDtypeStruct((n_mols, HIDDEN), jnp.float32),
    )(f_atoms, S, W_oa, W_lo, W_hi, b_o.reshape(1, HIDDEN), pool)


def kernel(f_atoms, f_bonds, a2b, b2a, b2revb, a_scope, W_i, b_i, W_h, b_h, W_o, b_o):
    n_mols = a_scope.shape[0]
    sizes = a_scope[:, 1]
    seg_ids = jnp.repeat(jnp.arange(n_mols, dtype=jnp.int32), sizes,
                         total_repeat_length=N_ATOMS)
    pool = (seg_ids[:, None] == jnp.arange(n_mols, dtype=jnp.int32)[None, :])
    pool = pool.astype(jnp.float32) / sizes[None, :].astype(jnp.float32)

    n_pad = N_ATOMS_PAD - N_ATOMS
    pad_idx = (jnp.arange(n_pad * 16, dtype=jnp.int32) * 37) % N_BONDS
    a2b_flat = jnp.concatenate([a2b.reshape(-1), pad_idx])

    W_lo = W_h[:HALF].astype(jnp.bfloat16)
    W_hi = W_h[HALF:].astype(jnp.bfloat16)
    NH = N_BONDS // 2
    pad_a = (jnp.arange(128, dtype=jnp.int32) * 7) % N_ATOMS
    pad_r = (jnp.arange(128, dtype=jnp.int32) * 41) % N_BONDS
    b2a0 = jnp.concatenate([b2a[:NH], pad_a])
    b2a1 = jnp.concatenate([b2a[NH:], pad_a])
    b2r0 = jnp.concatenate([b2revb[:NH], pad_r])
    b2r1 = jnp.concatenate([b2revb[NH:], pad_r])
    dead = jnp.zeros((N_BONDS, HALF), jnp.int32)
    inp, M = _k0(f_bonds, W_i, b_i)
    for _ in range(DEPTH - 1):
        S = _sc_segsum(M, a2b_flat)
        D0 = _sc_diff(S, M, b2a0, b2r0)
        D1 = _sc_diff(S, M, b2a1, b2r1)
        Ma = _k3r(D0, inp, dead, W_lo, W_hi, b_h, 0)
        Mn = _k3r(D1, inp, Ma, W_lo, W_hi, b_h, 1)
        dead, M = M, Mn
    S = _sc_segsum(M, a2b_flat)
    return _k4(f_atoms, S, W_o, b_o, pool)


# speedup vs baseline: 2.1469x; 1.0485x over previous
import dataclasses
import functools

import jax
import jax.numpy as jnp
from jax import lax
from jax.experimental import pallas as pl
from jax.experimental.pallas import tpu as pltpu
from jax.experimental.pallas import tpu_sc as plsc

N_BONDS = 160000
N_ATOMS = 10000
HIDDEN = 1024
HALF = HIDDEN // 2
DEPTH = 8

NW = 32
N_ATOMS_PAD = 10240
APW = N_ATOMS_PAD // NW
BPW = N_BONDS // NW
SEG_C = 4
DIF_C = 32

BB = 2000
AB = 2000


def _sc_compiler_params():
    cp = pltpu.CompilerParams()
    if "needs_layout_passes" in pltpu.CompilerParams.__dataclass_fields__:
        cp = dataclasses.replace(cp, needs_layout_passes=False)
    return cp


_MASK_HI = -65536


def _tc_pack_halves(lo_f, hi_f):
    lo = lax.bitcast_convert_type(lo_f, jnp.int32)
    hi = lax.bitcast_convert_type(hi_f, jnp.int32)
    lo_w = lax.shift_right_logical(lo + 32768, 16)
    hi_w = (hi + 32768) & _MASK_HI
    return lo_w | hi_w


def _tc_pack(x):
    return _tc_pack_halves(x[:, :HALF], x[:, HALF:])


def _tc_unpack_f32(w):
    lo = lax.bitcast_convert_type(lax.shift_left(w, 16), jnp.float32)
    hi = lax.bitcast_convert_type(w & _MASK_HI, jnp.float32)
    return lo, hi


def _tc_unpack_bf16(w):
    lo, hi = _tc_unpack_f32(w)
    return lo.astype(jnp.bfloat16), hi.astype(jnp.bfloat16)


def _sum16(rows_ref, stage_ref, n_atoms):
    @pl.loop(0, HALF, step=16)
    def _(c):
        for a in range(n_atoms):
            acc_a, acc_b = plsc.unpack(
                plsc.bitcast(rows_ref[a * 16, pl.ds(c, 16)], jnp.bfloat16),
                format=plsc.PackFormat.INTERLEAVED,
                preferred_element_type=jnp.float32)
            for i in range(1, 16):
                x_a, x_b = plsc.unpack(
                    plsc.bitcast(rows_ref[a * 16 + i, pl.ds(c, 16)],
                                 jnp.bfloat16),
                    format=plsc.PackFormat.INTERLEAVED,
                    preferred_element_type=jnp.float32)
                acc_a = acc_a + x_a
                acc_b = acc_b + x_b
            stage_ref[a, pl.ds(c, 16)] = plsc.bitcast(
                plsc.pack(acc_a, acc_b, format=plsc.PackFormat.INTERLEAVED),
                jnp.int32)


def _sc_segsum(M, a2b_flat):
    mesh = plsc.VectorSubcoreMesh(core_axis_name="c", subcore_axis_name="s")
    n_pairs = APW // SEG_C // 2

    @functools.partial(
        pl.kernel, mesh=mesh, compiler_params=_sc_compiler_params(),
        out_type=jax.ShapeDtypeStruct((N_ATOMS_PAD, HALF), jnp.int32),
        scratch_types=[
            pltpu.VMEM((APW * 16,), jnp.int32),
            pltpu.VMEM((SEG_C * 16, HALF), jnp.int32),
            pltpu.VMEM((SEG_C * 16, HALF), jnp.int32),
            pltpu.VMEM((SEG_C, HALF), jnp.int32),
            pltpu.VMEM((SEG_C, HALF), jnp.int32),
            pltpu.SemaphoreType.DMA,
            pltpu.SemaphoreType.DMA,
            pltpu.SemaphoreType.DMA,
            pltpu.SemaphoreType.DMA,
        ],
    )
    def k(m_hbm, idx_hbm, out_hbm, idx_v, r0, r1, st0, st1, g0, g1, s0, s1):
        wid = lax.axis_index("s") * 2 + lax.axis_index("c")
        pltpu.sync_copy(idx_hbm.at[pl.ds(wid * APW * 16, APW * 16)], idx_v)
        abase = wid * APW

        def gather(j, dst, sem):
            return pltpu.async_copy(
                m_hbm.at[idx_v.at[pl.ds(j * (SEG_C * 16), SEG_C * 16)]], dst, sem)

        def gwait(j, dst, sem):
            pltpu.make_async_copy(
                m_hbm.at[idx_v.at[pl.ds(j * (SEG_C * 16), SEG_C * 16)]], dst,
                sem).wait()

        def store(j, src, sem):
            return pltpu.async_copy(
                src, out_hbm.at[pl.ds(abase + j * SEG_C, SEG_C)], sem)

        def swait(src, sem):
            pltpu.make_async_copy(
                src, out_hbm.at[pl.ds(abase, SEG_C)], sem).wait()

        gather(0, r0, g0)
        gather(1, r1, g1)

        @pl.loop(0, n_pairs)
        def _(kk):
            j0 = 2 * kk
            j1 = j0 + 1

            @pl.when(kk > 0)
            def _():
                swait(st1, s1)
                gather(j1, r1, g1)

            gwait(j0, r0, g0)
            _sum16(r0, st0, SEG_C)
            store(j0, st0, s0)

            gwait(j1, r1, g1)
            _sum16(r1, st1, SEG_C)
            store(j1, st1, s1)

            @pl.when(kk < n_pairs - 1)
            def _():
                swait(st0, s0)
                gather(j0 + 2, r0, g0)

        swait(st0, s0)
        swait(st1, s1)

    return k(M, a2b_flat)


def _sc_diff(S, M, b2a, b2revb):
    mesh = plsc.VectorSubcoreMesh(core_axis_name="c", subcore_axis_name="s")
    nb = b2a.shape[0]
    bpw = nb // NW
    n_full = bpw // DIF_C
    tail = bpw % DIF_C
    n_pairs = n_full // 2
    assert bpw % 8 == 0 and n_full % 2 == 0 and 0 < tail < DIF_C, (nb, bpw)

    @functools.partial(
        pl.kernel, mesh=mesh, compiler_params=_sc_compiler_params(),
        out_type=jax.ShapeDtypeStruct((nb, HALF), jnp.int32),
        scratch_types=[
            pltpu.VMEM((bpw,), jnp.int32),
            pltpu.VMEM((bpw,), jnp.int32),
            pltpu.VMEM((DIF_C, HALF), jnp.int32),
            pltpu.VMEM((DIF_C, HALF), jnp.int32),
            pltpu.VMEM((DIF_C, HALF), jnp.int32),
            pltpu.VMEM((DIF_C, HALF), jnp.int32),
            pltpu.SemaphoreType.DMA,
            pltpu.SemaphoreType.DMA,
            pltpu.SemaphoreType.DMA,
            pltpu.SemaphoreType.DMA,
            pltpu.SemaphoreType.DMA,
            pltpu.SemaphoreType.DMA,
        ],
    )
    def k(s_hbm, m_hbm, ia_hbm, ir_hbm, out_hbm,
          ia_v, ir_v, sa0, sa1, mb0, mb1, ga0, ga1, gm0, gm1, ss0, ss1):
        wid = lax.axis_index("s") * 2 + lax.axis_index("c")
        pltpu.sync_copy(ia_hbm.at[pl.ds(wid * bpw, bpw)], ia_v)
        pltpu.sync_copy(ir_hbm.at[pl.ds(wid * bpw, bpw)], ir_v)
        bbase = wid * bpw

        def gathers(j, sdst, mdst, gsa, gsm):
            pltpu.async_copy(s_hbm.at[ia_v.at[pl.ds(j * DIF_C, DIF_C)]], sdst, gsa)
            pltpu.async_copy(m_hbm.at[ir_v.at[pl.ds(j * DIF_C, DIF_C)]], mdst, gsm)

        def gwaits(j, sdst, mdst, gsa, gsm):
            pltpu.make_async_copy(
                s_hbm.at[ia_v.at[pl.ds(j * DIF_C, DIF_C)]], sdst, gsa).wait()
            pltpu.make_async_copy(
                m_hbm.at[ir_v.at[pl.ds(j * DIF_C, DIF_C)]], mdst, gsm).wait()

        def sub_inplace(sdst, mdst, nrows=DIF_C):
            @pl.loop(0, HALF, step=16)
            def _(c):
                for r in range(nrows):
                    a = plsc.bitcast(sdst[r, pl.ds(c, 16)], jnp.bfloat16)
                    b = plsc.bitcast(mdst[r, pl.ds(c, 16)], jnp.bfloat16)
                    sdst[r, pl.ds(c, 16)] = plsc.bitcast(a - b, jnp.int32)

        def store(j, src, sem):
            pltpu.async_copy(src, out_hbm.at[pl.ds(bbase + j * DIF_C, DIF_C)], sem)

        def swait(src, sem):
            pltpu.make_async_copy(
                src, out_hbm.at[pl.ds(bbase, DIF_C)], sem).wait()

        gathers(0, sa0, mb0, ga0, gm0)
        gathers(1, sa1, mb1, ga1, gm1)

        @pl.loop(0, n_pairs)
        def _(kk):
            j0 = 2 * kk
            j1 = j0 + 1

            @pl.when(kk > 0)
            def _():
                swait(sa1, ss1)
                gathers(j1, sa1, mb1, ga1, gm1)

            gwaits(j0, sa0, mb0, ga0, gm0)
            sub_inplace(sa0, mb0)
            store(j0, sa0, ss0)

            gwaits(j1, sa1, mb1, ga1, gm1)
            sub_inplace(sa1, mb1)
            store(j1, sa1, ss1)

            swait(sa0, ss0)

            @pl.when(kk < n_pairs - 1)
            def _():
                gathers(j0 + 2, sa0, mb0, ga0, gm0)

        tbase = n_full * DIF_C
        sa0t = sa0.at[pl.ds(0, tail)]
        mb0t = mb0.at[pl.ds(0, tail)]
        pltpu.async_copy(s_hbm.at[ia_v.at[pl.ds(tbase, tail)]], sa0t, ga0)
        pltpu.async_copy(m_hbm.at[ir_v.at[pl.ds(tbase, tail)]], mb0t, gm0)
        pltpu.make_async_copy(
            s_hbm.at[ia_v.at[pl.ds(tbase, tail)]], sa0t, ga0).wait()
        pltpu.make_async_copy(
            m_hbm.at[ir_v.at[pl.ds(tbase, tail)]], mb0t, gm0).wait()
        sub_inplace(sa0, mb0, nrows=tail)
        pltpu.async_copy(sa0t, out_hbm.at[pl.ds(bbase + tbase, tail)], ss0)
        pltpu.make_async_copy(
            sa0t, out_hbm.at[pl.ds(bbase + tbase, tail)], ss0).wait()
        swait(sa1, ss1)

    return k(S, M, b2a, b2revb)


def _k0_body(fb_ref, wi_ref, bi_ref, inp_ref, m_ref):
    x = jnp.dot(fb_ref[...], wi_ref[...], preferred_element_type=jnp.float32)
    x = x + bi_ref[...]
    inp_ref[...] = _tc_pack(x)
    m_ref[...] = _tc_pack(jnp.maximum(x, 0.0))


def _k0(f_bonds, W_i, b_i):
    nfb = f_bonds.shape[1]
    grid = (N_BONDS // BB,)
    return pl.pallas_call(
        _k0_body,
        grid=grid,
        in_specs=[
            pl.BlockSpec((BB, nfb), lambda i: (i, 0)),
            pl.BlockSpec((nfb, HIDDEN), lambda i: (0, 0)),
            pl.BlockSpec((1, HIDDEN), lambda i: (0, 0)),
        ],
        out_specs=[
            pl.BlockSpec((BB, HALF), lambda i: (i, 0)),
            pl.BlockSpec((BB, HALF), lambda i: (i, 0)),
        ],
        out_shape=[
            jax.ShapeDtypeStruct((N_BONDS, HALF), jnp.int32),
            jax.ShapeDtypeStruct((N_BONDS, HALF), jnp.int32),
        ],
    )(f_bonds, W_i, b_i.reshape(1, HIDDEN))


def _k3_body(d_ref, inp_ref, wlo_ref, whi_ref, bh_ref, m_ref):
    lo, hi = _tc_unpack_bf16(d_ref[...])
    x = jnp.dot(lo, wlo_ref[...], preferred_element_type=jnp.float32)
    x += jnp.dot(hi, whi_ref[...], preferred_element_type=jnp.float32)
    ilo, ihi = _tc_unpack_f32(inp_ref[...])
    b = bh_ref[...]
    m_lo = jnp.maximum(x[:, :HALF] + ilo + b[:, :HALF], 0.0)
    m_hi = jnp.maximum(x[:, HALF:] + ihi + b[:, HALF:], 0.0)
    m_ref[...] = _tc_pack_halves(m_lo, m_hi)


def _k3_body_dead(d_ref, inp_ref, wlo_ref, whi_ref, bh_ref, dead_ref, m_ref):
    _k3_body(d_ref, inp_ref, wlo_ref, whi_ref, bh_ref, m_ref)


def _k3r(D, inp, dead, W_lo, W_hi, b_h, half):
    NH = N_BONDS // 2
    off = half * (NH // BB)
    grid = (NH // BB,)
    return pl.pallas_call(
        _k3_body_dead,
        grid=grid,
        in_specs=[
            pl.BlockSpec((BB, HALF), lambda i: (i, 0)),
            pl.BlockSpec((BB, HALF), lambda i: (i + off, 0)),
            pl.BlockSpec((HALF, HIDDEN), lambda i: (0, 0)),
            pl.BlockSpec((HALF, HIDDEN), lambda i: (0, 0)),
            pl.BlockSpec((1, HIDDEN), lambda i: (0, 0)),
            pl.BlockSpec(memory_space=pl.ANY),
        ],
        out_specs=pl.BlockSpec((BB, HALF), lambda i: (i + off, 0)),
        out_shape=jax.ShapeDtypeStruct((N_BONDS, HALF), jnp.int32),
        input_output_aliases={5: 0},
    )(D, inp, W_lo, W_hi, b_h.reshape(1, HIDDEN), dead)


def _k4_body(fa_ref, s_ref, woa_ref, wlo_ref, whi_ref, bo_ref, pool_ref,
             out_ref):
    h = jnp.dot(fa_ref[...], woa_ref[...], preferred_element_type=jnp.float32)
    lo, hi = _tc_unpack_bf16(s_ref[...])
    h += jnp.dot(lo, wlo_ref[...], preferred_element_type=jnp.float32)
    h += jnp.dot(hi, whi_ref[...], preferred_element_type=jnp.float32)
    h = jnp.maximum(h + bo_ref[...], 0.0)
    contrib = jax.lax.dot_general(
        pool_ref[...], h, (((0,), (0,)), ((), ())),
        preferred_element_type=jnp.float32)

    @pl.when(pl.program_id(0) == 0)
    def _():
        out_ref[...] = jnp.zeros_like(out_ref)

    out_ref[...] += contrib


def _k4(f_atoms, S, W_o, b_o, pool):
    nfa = f_atoms.shape[1]
    n_mols = pool.shape[1]
    W_oa = W_o[:nfa]
    W_lo = W_o[nfa:nfa + HALF].astype(jnp.bfloat16)
    W_hi = W_o[nfa + HALF:].astype(jnp.bfloat16)
    grid = (N_ATOMS // AB,)
    return pl.pallas_call(
        _k4_body,
        grid=grid,
        in_specs=[
            pl.BlockSpec((AB, nfa), lambda i: (i, 0)),
            pl.BlockSpec((AB, HALF), lambda i: (i, 0)),
            pl.BlockSpec((nfa, HIDDEN), lambda i: (0, 0)),
            pl.BlockSpec((HALF, HIDDEN), lambda i: (0, 0)),
            pl.BlockSpec((HALF, HIDDEN), lambda i: (0, 0)),
            pl.BlockSpec((1, HIDDEN), lambda i: (0, 0)),
            pl.BlockSpec((AB, n_mols), lambda i: (i, 0)),
        ],
        out_specs=pl.BlockSpec((n_mols, HIDDEN), lambda i: (0, 0)),
        out_shape=jax.ShapeDtypeStruct((n_mols, HIDDEN), jnp.float32),
    )(f_atoms, S, W_oa, W_lo, W_hi, b_o.reshape(1, HIDDEN), pool)


def kernel(f_atoms, f_bonds, a2b, b2a, b2revb, a_scope, W_i, b_i, W_h, b_h, W_o, b_o):
    n_mols = a_scope.shape[0]
    sizes = a_scope[:, 1]
    seg_ids = jnp.repeat(jnp.arange(n_mols, dtype=jnp.int32), sizes,
                         total_repeat_length=N_ATOMS)
    pool = (seg_ids[:, None] == jnp.arange(n_mols, dtype=jnp.int32)[None, :])
    pool = pool.astype(jnp.float32) / sizes[None, :].astype(jnp.float32)

    n_pad = N_ATOMS_PAD - N_ATOMS
    pad_idx = (jnp.arange(n_pad * 16, dtype=jnp.int32) * 37) % N_BONDS
    a2b_flat = jnp.concatenate([a2b.reshape(-1), pad_idx])

    W_lo = W_h[:HALF].astype(jnp.bfloat16)
    W_hi = W_h[HALF:].astype(jnp.bfloat16)
    NH = N_BONDS // 2
    pad_a = (jnp.arange(128, dtype=jnp.int32) * 7) % N_ATOMS
    pad_r = (jnp.arange(128, dtype=jnp.int32) * 41) % N_BONDS
    b2a0 = jnp.concatenate([b2a[:NH], pad_a])
    b2a1 = jnp.concatenate([b2a[NH:], pad_a])
    b2r0 = jnp.concatenate([b2revb[:NH], pad_r])
    b2r1 = jnp.concatenate([b2revb[NH:], pad_r])
    dead = jnp.zeros((N_BONDS, HALF), jnp.int32)
    inp, M = _k0(f_bonds, W_i, b_i)
    for _ in range(DEPTH - 1):
        S = _sc_segsum(M, a2b_flat)
        D0 = _sc_diff(S, M, b2a0, b2r0)
        D1 = _sc_diff(S, M, b2a1, b2r1)
        Ma = _k3r(D0, inp, dead, W_lo, W_hi, b_h, 0)
        Mn = _k3r(D1, inp, Ma, W_lo, W_hi, b_h, 1)
        dead, M = M, Mn
    S = _sc_segsum(M, a2b_flat)
    return _k4(f_atoms, S, W_o, b_o, pool)


# speedup vs baseline: 2.2519x; 1.0489x over previous
import dataclasses
import functools

import jax
import jax.numpy as jnp
from jax import lax
from jax.experimental import pallas as pl
from jax.experimental.pallas import tpu as pltpu
from jax.experimental.pallas import tpu_sc as plsc

N_BONDS = 160000
N_ATOMS = 10000
HIDDEN = 1024
HALF = HIDDEN // 2
DEPTH = 8

NW = 32
N_ATOMS_PAD = 10240
APW = N_ATOMS_PAD // NW
BPW = N_BONDS // NW
SEG_C = 4
DIF_C = 32

BB = 2000
AB = 2000


def _sc_compiler_params():
    cp = pltpu.CompilerParams()
    if "needs_layout_passes" in pltpu.CompilerParams.__dataclass_fields__:
        cp = dataclasses.replace(cp, needs_layout_passes=False)
    return cp


_MASK_HI = -65536


def _tc_pack_halves(lo_f, hi_f):
    lo = lax.bitcast_convert_type(lo_f, jnp.int32)
    hi = lax.bitcast_convert_type(hi_f, jnp.int32)
    lo_w = lax.shift_right_logical(lo + 32768, 16)
    hi_w = (hi + 32768) & _MASK_HI
    return lo_w | hi_w


def _tc_pack(x):
    return _tc_pack_halves(x[:, :HALF], x[:, HALF:])


def _tc_unpack_f32(w):
    lo = lax.bitcast_convert_type(lax.shift_left(w, 16), jnp.float32)
    hi = lax.bitcast_convert_type(w & _MASK_HI, jnp.float32)
    return lo, hi


def _tc_unpack_bf16(w):
    lo, hi = _tc_unpack_f32(w)
    return lo.astype(jnp.bfloat16), hi.astype(jnp.bfloat16)


def _sum16(rows_ref, stage_ref, n_atoms):
    @pl.loop(0, HALF, step=16)
    def _(c):
        for a in range(n_atoms):
            acc_a, acc_b = plsc.unpack(
                plsc.bitcast(rows_ref[a * 16, pl.ds(c, 16)], jnp.bfloat16),
                format=plsc.PackFormat.INTERLEAVED,
                preferred_element_type=jnp.float32)
            for i in range(1, 16):
                x_a, x_b = plsc.unpack(
                    plsc.bitcast(rows_ref[a * 16 + i, pl.ds(c, 16)],
                                 jnp.bfloat16),
                    format=plsc.PackFormat.INTERLEAVED,
                    preferred_element_type=jnp.float32)
                acc_a = acc_a + x_a
                acc_b = acc_b + x_b
            stage_ref[a, pl.ds(c, 16)] = plsc.bitcast(
                plsc.pack(acc_a, acc_b, format=plsc.PackFormat.INTERLEAVED),
                jnp.int32)


def _sc_segsum(M, a2b_flat):
    mesh = plsc.VectorSubcoreMesh(core_axis_name="c", subcore_axis_name="s")
    n_pairs = APW // SEG_C // 2

    @functools.partial(
        pl.kernel, mesh=mesh, compiler_params=_sc_compiler_params(),
        out_type=jax.ShapeDtypeStruct((N_ATOMS_PAD, HALF), jnp.int32),
        scratch_types=[
            pltpu.VMEM((APW * 16,), jnp.int32),
            pltpu.VMEM((SEG_C * 16, HALF), jnp.int32),
            pltpu.VMEM((SEG_C * 16, HALF), jnp.int32),
            pltpu.VMEM((SEG_C, HALF), jnp.int32),
            pltpu.VMEM((SEG_C, HALF), jnp.int32),
            pltpu.SemaphoreType.DMA,
            pltpu.SemaphoreType.DMA,
            pltpu.SemaphoreType.DMA,
            pltpu.SemaphoreType.DMA,
        ],
    )
    def k(m_hbm, idx_hbm, out_hbm, idx_v, r0, r1, st0, st1, g0, g1, s0, s1):
        wid = lax.axis_index("s") * 2 + lax.axis_index("c")
        pltpu.sync_copy(idx_hbm.at[pl.ds(wid * APW * 16, APW * 16)], idx_v)
        abase = wid * APW

        def gather(j, dst, sem):
            return pltpu.async_copy(
                m_hbm.at[idx_v.at[pl.ds(j * (SEG_C * 16), SEG_C * 16)]], dst, sem)

        def gwait(j, dst, sem):
            pltpu.make_async_copy(
                m_hbm.at[idx_v.at[pl.ds(j * (SEG_C * 16), SEG_C * 16)]], dst,
                sem).wait()

        def store(j, src, sem):
            return pltpu.async_copy(
                src, out_hbm.at[pl.ds(abase + j * SEG_C, SEG_C)], sem)

        def swait(src, sem):
            pltpu.make_async_copy(
                src, out_hbm.at[pl.ds(abase, SEG_C)], sem).wait()

        gather(0, r0, g0)
        gather(1, r1, g1)

        @pl.loop(0, n_pairs)
        def _(kk):
            j0 = 2 * kk
            j1 = j0 + 1

            @pl.when(kk > 0)
            def _():
                swait(st1, s1)
                gather(j1, r1, g1)

            gwait(j0, r0, g0)
            _sum16(r0, st0, SEG_C)
            store(j0, st0, s0)

            gwait(j1, r1, g1)
            _sum16(r1, st1, SEG_C)
            store(j1, st1, s1)

            @pl.when(kk < n_pairs - 1)
            def _():
                swait(st0, s0)
                gather(j0 + 2, r0, g0)

        swait(st0, s0)
        swait(st1, s1)

    return k(M, a2b_flat)


def _sc_diff(S, M, b2a, b2revb, dif_c=DIF_C):
    mesh = plsc.VectorSubcoreMesh(core_axis_name="c", subcore_axis_name="s")
    nb = b2a.shape[0]
    bpw = nb // NW
    n_full = bpw // dif_c
    tail = bpw % dif_c
    n_pairs = n_full // 2
    assert bpw % 8 == 0 and n_full % 2 == 0 and 0 < tail < dif_c, (nb, bpw)

    @functools.partial(
        pl.kernel, mesh=mesh, compiler_params=_sc_compiler_params(),
        out_type=jax.ShapeDtypeStruct((nb, HALF), jnp.int32),
        scratch_types=[
            pltpu.VMEM((bpw,), jnp.int32),
            pltpu.VMEM((bpw,), jnp.int32),
            pltpu.VMEM((dif_c, HALF), jnp.int32),
            pltpu.VMEM((dif_c, HALF), jnp.int32),
            pltpu.VMEM((dif_c, HALF), jnp.int32),
            pltpu.VMEM((dif_c, HALF), jnp.int32),
            pltpu.SemaphoreType.DMA,
            pltpu.SemaphoreType.DMA,
            pltpu.SemaphoreType.DMA,
            pltpu.SemaphoreType.DMA,
            pltpu.SemaphoreType.DMA,
            pltpu.SemaphoreType.DMA,
        ],
    )
    def k(s_hbm, m_hbm, ia_hbm, ir_hbm, out_hbm,
          ia_v, ir_v, sa0, sa1, mb0, mb1, ga0, ga1, gm0, gm1, ss0, ss1):
        wid = lax.axis_index("s") * 2 + lax.axis_index("c")
        pltpu.sync_copy(ia_hbm.at[pl.ds(wid * bpw, bpw)], ia_v)
        pltpu.sync_copy(ir_hbm.at[pl.ds(wid * bpw, bpw)], ir_v)
        bbase = wid * bpw

        def gathers(j, sdst, mdst, gsa, gsm):
            pltpu.async_copy(s_hbm.at[ia_v.at[pl.ds(j * dif_c, dif_c)]], sdst, gsa)
            pltpu.async_copy(m_hbm.at[ir_v.at[pl.ds(j * dif_c, dif_c)]], mdst, gsm)

        def gwaits(j, sdst, mdst, gsa, gsm):
            pltpu.make_async_copy(
                s_hbm.at[ia_v.at[pl.ds(j * dif_c, dif_c)]], sdst, gsa).wait()
            pltpu.make_async_copy(
                m_hbm.at[ir_v.at[pl.ds(j * dif_c, dif_c)]], mdst, gsm).wait()

        def sub_inplace(sdst, mdst, nrows=dif_c):
            @pl.loop(0, HALF, step=16)
            def _(c):
                for r in range(nrows):
                    a = plsc.bitcast(sdst[r, pl.ds(c, 16)], jnp.bfloat16)
                    b = plsc.bitcast(mdst[r, pl.ds(c, 16)], jnp.bfloat16)
                    sdst[r, pl.ds(c, 16)] = plsc.bitcast(a - b, jnp.int32)

        def store(j, src, sem):
            pltpu.async_copy(src, out_hbm.at[pl.ds(bbase + j * dif_c, dif_c)], sem)

        def swait(src, sem):
            pltpu.make_async_copy(
                src, out_hbm.at[pl.ds(bbase, dif_c)], sem).wait()

        gathers(0, sa0, mb0, ga0, gm0)
        gathers(1, sa1, mb1, ga1, gm1)

        @pl.loop(0, n_pairs)
        def _(kk):
            j0 = 2 * kk
            j1 = j0 + 1

            @pl.when(kk > 0)
            def _():
                swait(sa1, ss1)
                gathers(j1, sa1, mb1, ga1, gm1)

            gwaits(j0, sa0, mb0, ga0, gm0)
            sub_inplace(sa0, mb0)
            store(j0, sa0, ss0)

            gwaits(j1, sa1, mb1, ga1, gm1)
            sub_inplace(sa1, mb1)
            store(j1, sa1, ss1)

            swait(sa0, ss0)

            @pl.when(kk < n_pairs - 1)
            def _():
                gathers(j0 + 2, sa0, mb0, ga0, gm0)

        tbase = n_full * dif_c
        sa0t = sa0.at[pl.ds(0, tail)]
        mb0t = mb0.at[pl.ds(0, tail)]
        pltpu.async_copy(s_hbm.at[ia_v.at[pl.ds(tbase, tail)]], sa0t, ga0)
        pltpu.async_copy(m_hbm.at[ir_v.at[pl.ds(tbase, tail)]], mb0t, gm0)
        pltpu.make_async_copy(
            s_hbm.at[ia_v.at[pl.ds(tbase, tail)]], sa0t, ga0).wait()
        pltpu.make_async_copy(
            m_hbm.at[ir_v.at[pl.ds(tbase, tail)]], mb0t, gm0).wait()
        sub_inplace(sa0, mb0, nrows=tail)
        pltpu.async_copy(sa0t, out_hbm.at[pl.ds(bbase + tbase, tail)], ss0)
        pltpu.make_async_copy(
            sa0t, out_hbm.at[pl.ds(bbase + tbase, tail)], ss0).wait()
        swait(sa1, ss1)

    return k(S, M, b2a, b2revb)


def _k0_body(fb_ref, wi_ref, bi_ref, inp_ref, m_ref):
    x = jnp.dot(fb_ref[...], wi_ref[...], preferred_element_type=jnp.float32)
    x = x + bi_ref[...]
    inp_ref[...] = _tc_pack(x)
    m_ref[...] = _tc_pack(jnp.maximum(x, 0.0))


def _k0(f_bonds, W_i, b_i):
    nfb = f_bonds.shape[1]
    grid = (N_BONDS // BB,)
    return pl.pallas_call(
        _k0_body,
        grid=grid,
        in_specs=[
            pl.BlockSpec((BB, nfb), lambda i: (i, 0)),
            pl.BlockSpec((nfb, HIDDEN), lambda i: (0, 0)),
            pl.BlockSpec((1, HIDDEN), lambda i: (0, 0)),
        ],
        out_specs=[
            pl.BlockSpec((BB, HALF), lambda i: (i, 0)),
            pl.BlockSpec((BB, HALF), lambda i: (i, 0)),
        ],
        out_shape=[
            jax.ShapeDtypeStruct((N_BONDS, HALF), jnp.int32),
            jax.ShapeDtypeStruct((N_BONDS, HALF), jnp.int32),
        ],
    )(f_bonds, W_i, b_i.reshape(1, HIDDEN))


def _k3_body(d_ref, inp_ref, wlo_ref, whi_ref, bh_ref, m_ref):
    lo, hi = _tc_unpack_bf16(d_ref[...])
    x = jnp.dot(lo, wlo_ref[...], preferred_element_type=jnp.float32)
    x += jnp.dot(hi, whi_ref[...], preferred_element_type=jnp.float32)
    ilo, ihi = _tc_unpack_f32(inp_ref[...])
    b = bh_ref[...]
    m_lo = jnp.maximum(x[:, :HALF] + ilo + b[:, :HALF], 0.0)
    m_hi = jnp.maximum(x[:, HALF:] + ihi + b[:, HALF:], 0.0)
    m_ref[...] = _tc_pack_halves(m_lo, m_hi)


def _k3_body_dead(d_ref, inp_ref, wlo_ref, whi_ref, bh_ref, dead_ref, m_ref):
    _k3_body(d_ref, inp_ref, wlo_ref, whi_ref, bh_ref, m_ref)


def _k3r(D, inp, dead, W_lo, W_hi, b_h, off_blocks):
    off = off_blocks
    grid = (D.shape[0] // BB,)
    return pl.pallas_call(
        _k3_body_dead,
        grid=grid,
        in_specs=[
            pl.BlockSpec((BB, HALF), lambda i: (i, 0)),
            pl.BlockSpec((BB, HALF), lambda i: (i + off, 0)),
            pl.BlockSpec((HALF, HIDDEN), lambda i: (0, 0)),
            pl.BlockSpec((HALF, HIDDEN), lambda i: (0, 0)),
            pl.BlockSpec((1, HIDDEN), lambda i: (0, 0)),
            pl.BlockSpec(memory_space=pl.ANY),
        ],
        out_specs=pl.BlockSpec((BB, HALF), lambda i: (i + off, 0)),
        out_shape=jax.ShapeDtypeStruct((N_BONDS, HALF), jnp.int32),
        input_output_aliases={5: 0},
    )(D, inp, W_lo, W_hi, b_h.reshape(1, HIDDEN), dead)


def _k4_body(fa_ref, s_ref, woa_ref, wlo_ref, whi_ref, bo_ref, pool_ref,
             out_ref):
    h = jnp.dot(fa_ref[...], woa_ref[...], preferred_element_type=jnp.float32)
    lo, hi = _tc_unpack_bf16(s_ref[...])
    h += jnp.dot(lo, wlo_ref[...], preferred_element_type=jnp.float32)
    h += jnp.dot(hi, whi_ref[...], preferred_element_type=jnp.float32)
    h = jnp.maximum(h + bo_ref[...], 0.0)
    contrib = jax.lax.dot_general(
        pool_ref[...], h, (((0,), (0,)), ((), ())),
        preferred_element_type=jnp.float32)

    @pl.when(pl.program_id(0) == 0)
    def _():
        out_ref[...] = jnp.zeros_like(out_ref)

    out_ref[...] += contrib


def _k4(f_atoms, S, W_o, b_o, pool):
    nfa = f_atoms.shape[1]
    n_mols = pool.shape[1]
    W_oa = W_o[:nfa]
    W_lo = W_o[nfa:nfa + HALF].astype(jnp.bfloat16)
    W_hi = W_o[nfa + HALF:].astype(jnp.bfloat16)
    grid = (N_ATOMS // AB,)
    return pl.pallas_call(
        _k4_body,
        grid=grid,
        in_specs=[
            pl.BlockSpec((AB, nfa), lambda i: (i, 0)),
            pl.BlockSpec((AB, HALF), lambda i: (i, 0)),
            pl.BlockSpec((nfa, HIDDEN), lambda i: (0, 0)),
            pl.BlockSpec((HALF, HIDDEN), lambda i: (0, 0)),
            pl.BlockSpec((HALF, HIDDEN), lambda i: (0, 0)),
            pl.BlockSpec((1, HIDDEN), lambda i: (0, 0)),
            pl.BlockSpec((AB, n_mols), lambda i: (i, 0)),
        ],
        out_specs=pl.BlockSpec((n_mols, HIDDEN), lambda i: (0, 0)),
        out_shape=jax.ShapeDtypeStruct((n_mols, HIDDEN), jnp.float32),
    )(f_atoms, S, W_oa, W_lo, W_hi, b_o.reshape(1, HIDDEN), pool)


def kernel(f_atoms, f_bonds, a2b, b2a, b2revb, a_scope, W_i, b_i, W_h, b_h, W_o, b_o):
    n_mols = a_scope.shape[0]
    sizes = a_scope[:, 1]
    seg_ids = jnp.repeat(jnp.arange(n_mols, dtype=jnp.int32), sizes,
                         total_repeat_length=N_ATOMS)
    pool = (seg_ids[:, None] == jnp.arange(n_mols, dtype=jnp.int32)[None, :])
    pool = pool.astype(jnp.float32) / sizes[None, :].astype(jnp.float32)

    n_pad = N_ATOMS_PAD - N_ATOMS
    pad_idx = (jnp.arange(n_pad * 16, dtype=jnp.int32) * 37) % N_BONDS
    a2b_flat = jnp.concatenate([a2b.reshape(-1), pad_idx])

    W_lo = W_h[:HALF].astype(jnp.bfloat16)
    W_hi = W_h[HALF:].astype(jnp.bfloat16)
    parts = [(0, 64000, 32), (64000, 64000, 32), (128000, 32000, 16)]
    dead = jnp.zeros((N_BONDS, HALF), jnp.int32)
    inp, M = _k0(f_bonds, W_i, b_i)
    for _ in range(DEPTH - 1):
        S = _sc_segsum(M, a2b_flat)
        Ds = [_sc_diff(S, M, b2a[o:o + n], b2revb[o:o + n], dc)
              for (o, n, dc) in parts]
        cur = dead
        for (o, n, dc), Dp in zip(parts, Ds):
            cur = _k3r(Dp, inp, cur, W_lo, W_hi, b_h, o // BB)
        dead, M = M, cur
    S = _sc_segsum(M, a2b_flat)
    return _k4(f_atoms, S, W_o, b_o, pool)
